# Initial kernel scaffold; baseline (speedup 1.0000x reference)
import jax
import jax.numpy as jnp
from jax.experimental import pallas as pl


def kernel(x, edge_index, e, W1n, b1n, W2n, b2n, W1e, b1e, W2e, b2e, gA1, bgA1, gA2, bgA2, gA3, bgA3, gB1, bgB1, gB2, bgB2, gB3, bgB3, gam_h, bet_h, gam_e, bet_e, pW1, pb1, pW2, pb2):
    raise NotImplementedError("write your pallas kernel here")


# speedup vs baseline: 1.0431x; 1.0431x over previous
import jax
import jax.numpy as jnp
from jax.experimental import pallas as pl

L = 8


def _bn(t, gamma, beta, eps=1e-5):
    mu = jnp.mean(t, axis=0, keepdims=True)
    var = jnp.var(t, axis=0, keepdims=True)
    return gamma * (t - mu) / jnp.sqrt(var + eps) + beta


def _final_block(ph_ref, w_ref, b_ref, o_ref):
    o_ref[...] = ph_ref[...] @ w_ref[...] + b_ref[...]


def kernel(x, edge_index, e, W1n, b1n, W2n, b2n, W1e, b1e, W2e, b2e,
           gA1, bgA1, gA2, bgA2, gA3, bgA3, gB1, bgB1, gB2, bgB2, gB3, bgB3,
           gam_h, bet_h, gam_e, bet_e, pW1, pb1, pW2, pb2):
    src = edge_index[0]
    dst = edge_index[1]
    N = x.shape[0]
    h = jnp.maximum(x @ W1n + b1n, 0.0)
    h = h @ W2n + b2n
    x2 = jax.random.normal(jax.random.key(1), h.shape, dtype=h.dtype)
    data = jnp.concatenate([h[src], h[dst], x2[src], x2[dst]], axis=1)
    he = jnp.maximum(data @ W1e + b1e, 0.0)
    ef = jnp.maximum(he @ W2e + b2e, 0.0)
    for l in range(L):
        A1h = h @ gA1[l] + bgA1[l]
        A2h = h @ gA2[l] + bgA2[l]
        A3h = h @ gA3[l] + bgA3[l]
        B1h = h @ gB1[l] + bgB1[l]
        B2h = h @ gB2[l] + bgB2[l]
        B3e = ef @ gB3[l] + bgB3[l]
        e_tmp = B1h[src] + B2h[dst] + B3e
        ef = ef + jnp.maximum(_bn(e_tmp, gam_e[l], bet_e[l]), 0.0)
        sigma = jax.nn.sigmoid(ef)
        num_f = jax.ops.segment_sum(sigma * A2h[src], dst, num_segments=N)
        den_f = jax.ops.segment_sum(sigma, dst, num_segments=N)
        h_f = num_f / (den_f + 1e-6)
        num_b = jax.ops.segment_sum(sigma * A3h[dst], src, num_segments=N)
        den_b = jax.ops.segment_sum(sigma, src, num_segments=N)
        h_b = num_b / (den_b + 1e-6)
        h = h + jnp.maximum(_bn(A1h + h_f + h_b, gam_h[l], bet_h[l]), 0.0)
    pdata = jnp.concatenate([h[src], h[dst], ef], axis=1)
    ph = jnp.maximum(pdata @ pW1 + pb1, 0.0)
    E = ph.shape[0]
    BLK = 8000
    scores = pl.pallas_call(
        _final_block,
        grid=(E // BLK,),
        in_specs=[
            pl.BlockSpec((BLK, 64), lambda i: (i, 0)),
            pl.BlockSpec((64, 1), lambda i: (0, 0)),
            pl.BlockSpec((1,), lambda i: (0,)),
        ],
        out_specs=pl.BlockSpec((BLK, 1), lambda i: (i, 0)),
        out_shape=jax.ShapeDtypeStruct((E, 1), jnp.float32),
    )(ph, pW2, pb2)
    return scores


# speedup vs baseline: 3.6534x; 3.5023x over previous
import functools

import jax
import jax.numpy as jnp
from jax import lax
from jax.experimental import pallas as pl
from jax.experimental.pallas import tpu as pltpu
from jax.experimental.pallas import tpu_sc as plsc

L = 8
N = 10000
E = 320000
D = 64
D2 = 128
NC = 2
NS = 16
NW = NC * NS
EPW = E // NW
SUB = 40
KSUB = 5
CHUNK = SUB * KSUB
NCHUNK = EPW // CHUNK
CHUNKB = 80
NCHUNKB = EPW // CHUNKB

_mesh = plsc.VectorSubcoreMesh(core_axis_name="c", subcore_axis_name="s")
_f32 = jnp.float32


def _wid():
    return lax.axis_index("s") * NC + lax.axis_index("c")


def _sc_pass_a(src_h, dst_h, b3e_h, t12_h, etmp_h, stats_h,
               idx1_v, idx2_v, r1_v, r2_v, acc_v, st_v, sem):
    wid = _wid()
    ebase = wid * EPW

    def chunk_body(ci, carry):
        off = ebase + ci * CHUNK
        pltpu.sync_copy(src_h.at[pl.ds(off, CHUNK)], idx1_v)
        pltpu.sync_copy(dst_h.at[pl.ds(off, CHUNK)], idx2_v)
        cps = []
        for k in range(KSUB):
            sl = pl.ds(SUB * k, SUB)
            cps.append(pltpu.async_copy(
                t12_h.at[idx1_v.at[sl]], r1_v.at[sl], sem))
            cps.append(pltpu.async_copy(
                t12_h.at[idx2_v.at[sl]], r2_v.at[sl], sem))
        pltpu.sync_copy(b3e_h.at[pl.ds(off, CHUNK)], acc_v)
        for cp in cps:
            cp.wait()

        def row(r, c2):
            sums = list(c2)
            for j in range(4):
                sl = pl.ds(16 * j, 16)
                sh = pl.ds(64 + 16 * j, 16)
                a = acc_v[r, sl] + r1_v[r, sl] + r2_v[r, sh]
                acc_v[r, sl] = a
                sums[j] = sums[j] + a
                sums[4 + j] = sums[4 + j] + a * a
            return tuple(sums)

        carry = lax.fori_loop(0, CHUNK, row, carry)
        pltpu.sync_copy(acc_v, etmp_h.at[pl.ds(off, CHUNK)])
        return carry

    z = jnp.zeros((16,), _f32)
    sums = lax.fori_loop(0, NCHUNK, chunk_body, (z,) * 8)
    for j in range(4):
        st_v[0, pl.ds(16 * j, 16)] = sums[j]
        st_v[1, pl.ds(16 * j, 16)] = sums[4 + j]
    pltpu.sync_copy(st_v, stats_h.at[wid])


def _sc_pass_b(src_h, dst_h, etmp_h, ef_h, t23_h, st_h, zero_h,
               efn_h, nfd_h,
               idx1_v, idx2_v, idx2d_v, r1_v, et_v, ef_v, st_v, nfd_sh, sem):
    cid = lax.axis_index("c")
    sid = lax.axis_index("s")
    wid = sid * NC + cid
    ebase = wid * EPW

    @pl.when(sid == 0)
    def _():
        pltpu.sync_copy(zero_h, nfd_sh)
    plsc.subcore_barrier()

    pltpu.sync_copy(st_h, st_v)
    sv = [st_v[0, pl.ds(16 * j, 16)] for j in range(4)]
    tv = [st_v[1, pl.ds(16 * j, 16)] for j in range(4)]

    def chunk_body(ci, carry):
        off = ebase + ci * CHUNKB
        pltpu.sync_copy(src_h.at[pl.ds(off, CHUNKB)], idx1_v)
        pltpu.sync_copy(dst_h.at[pl.ds(off, CHUNKB)], idx2_v)
        cp = pltpu.async_copy(t23_h.at[idx1_v], r1_v, sem)
        pltpu.sync_copy(etmp_h.at[pl.ds(off, CHUNKB)], et_v)
        pltpu.sync_copy(ef_h.at[pl.ds(off, CHUNKB)], ef_v)
        cp.wait()

        def row(r, c2):
            for j in range(4):
                sl = pl.ds(16 * j, 16)
                sh = pl.ds(64 + 16 * j, 16)
                x = jnp.maximum(et_v[r, sl] * sv[j] + tv[j], 0.0) + ef_v[r, sl]
                ef_v[r, sl] = x
                sg = 1.0 / (1.0 + jnp.exp(-x))
                r1_v[r, sl] = sg * r1_v[r, sl]
                r1_v[r, sh] = sg
            return c2

        lax.fori_loop(0, CHUNKB, row, 0)
        pltpu.sync_copy(ef_v, efn_h.at[pl.ds(off, CHUNKB)])
        for m in range(CHUNKB // 16):
            idx2d_v[0, pl.ds(16 * m, 16)] = idx2_v[pl.ds(16 * m, 16)]
        pltpu.sync_copy(r1_v, nfd_sh.at[idx2d_v.at[0]], add=True)
        return carry

    lax.fori_loop(0, NCHUNKB, chunk_body, 0)
    plsc.subcore_barrier()

    @pl.when(sid == 0)
    def _():
        pltpu.sync_copy(nfd_sh, nfd_h.at[cid])


def _sc_pass_c(src_h, dst_h, efn_h, t23_h, zero_h,
               nbd_h,
               idx1_v, idx2_v, idx2d_v, r1_v, ef_v, nbd_sh, sem):
    cid = lax.axis_index("c")
    sid = lax.axis_index("s")
    wid = sid * NC + cid
    ebase = wid * EPW

    @pl.when(sid == 0)
    def _():
        pltpu.sync_copy(zero_h, nbd_sh)
    plsc.subcore_barrier()

    def chunk_body(ci, carry):
        off = ebase + ci * CHUNKB
        pltpu.sync_copy(src_h.at[pl.ds(off, CHUNKB)], idx1_v)
        pltpu.sync_copy(dst_h.at[pl.ds(off, CHUNKB)], idx2_v)
        cp = pltpu.async_copy(t23_h.at[idx2_v], r1_v, sem)
        pltpu.sync_copy(efn_h.at[pl.ds(off, CHUNKB)], ef_v)
        cp.wait()

        def row(r, c2):
            for j in range(4):
                sl = pl.ds(16 * j, 16)
                sh = pl.ds(64 + 16 * j, 16)
                sg = 1.0 / (1.0 + jnp.exp(-ef_v[r, sl]))
                r1_v[r, sl] = sg * r1_v[r, sh]
                r1_v[r, sh] = sg
            return c2

        lax.fori_loop(0, CHUNKB, row, 0)
        for m in range(CHUNKB // 16):
            idx2d_v[0, pl.ds(16 * m, 16)] = idx1_v[pl.ds(16 * m, 16)]
        pltpu.sync_copy(r1_v, nbd_sh.at[idx2d_v.at[0]], add=True)
        return carry

    lax.fori_loop(0, NCHUNKB, chunk_body, 0)
    plsc.subcore_barrier()

    @pl.when(sid == 0)
    def _():
        pltpu.sync_copy(nbd_sh, nbd_h.at[cid])


def _sc_gather2(src_h, dst_h, t_h, out_h,
                idx1_v, idx2_v, r1_v, r2_v, o_v, sem):
    wid = _wid()
    ebase = wid * EPW

    def chunk_body(ci, carry):
        off = ebase + ci * CHUNK
        pltpu.sync_copy(src_h.at[pl.ds(off, CHUNK)], idx1_v)
        pltpu.sync_copy(dst_h.at[pl.ds(off, CHUNK)], idx2_v)
        cps = []
        for k in range(KSUB):
            sl = pl.ds(SUB * k, SUB)
            cps.append(pltpu.async_copy(
                t_h.at[idx1_v.at[sl]], r1_v.at[sl], sem))
            cps.append(pltpu.async_copy(
                t_h.at[idx2_v.at[sl]], r2_v.at[sl], sem))
        for cp in cps:
            cp.wait()

        def row(r, c2):
            for j in range(4):
                sl = pl.ds(16 * j, 16)
                sh = pl.ds(64 + 16 * j, 16)
                o_v[r, sl] = jnp.maximum(r1_v[r, sl] + r2_v[r, sh], 0.0)
            return c2

        lax.fori_loop(0, CHUNK, row, 0)
        pltpu.sync_copy(o_v, out_h.at[pl.ds(off, CHUNK)])
        return carry

    lax.fori_loop(0, NCHUNK, chunk_body, 0)


def _sc_gather2e(src_h, dst_h, t_h, ex_h, out_h,
                 idx1_v, idx2_v, r1_v, r2_v, ex_v, sem):
    wid = _wid()
    ebase = wid * EPW

    def chunk_body(ci, carry):
        off = ebase + ci * CHUNK
        pltpu.sync_copy(src_h.at[pl.ds(off, CHUNK)], idx1_v)
        pltpu.sync_copy(dst_h.at[pl.ds(off, CHUNK)], idx2_v)
        cps = []
        for k in range(KSUB):
            sl = pl.ds(SUB * k, SUB)
            cps.append(pltpu.async_copy(
                t_h.at[idx1_v.at[sl]], r1_v.at[sl], sem))
            cps.append(pltpu.async_copy(
                t_h.at[idx2_v.at[sl]], r2_v.at[sl], sem))
        pltpu.sync_copy(ex_h.at[pl.ds(off, CHUNK)], ex_v)
        for cp in cps:
            cp.wait()

        def row(r, c2):
            for j in range(4):
                sl = pl.ds(16 * j, 16)
                sh = pl.ds(64 + 16 * j, 16)
                ex_v[r, sl] = jnp.maximum(
                    r1_v[r, sl] + r2_v[r, sh] + ex_v[r, sl], 0.0)
            return c2

        lax.fori_loop(0, CHUNK, row, 0)
        pltpu.sync_copy(ex_v, out_h.at[pl.ds(off, CHUNK)])
        return carry

    lax.fori_loop(0, NCHUNK, chunk_body, 0)


def _sds(shape):
    return jax.ShapeDtypeStruct(shape, _f32)


_IDX = pltpu.VMEM((CHUNK,), jnp.int32)
_ROWS = pltpu.VMEM((CHUNK, D2), _f32)
_HALF = pltpu.VMEM((CHUNK, D), _f32)
_IDXB = pltpu.VMEM((CHUNKB,), jnp.int32)
_IDX2DB = pltpu.VMEM((1, CHUNKB), jnp.int32)
_ROWSB = pltpu.VMEM((CHUNKB, D2), _f32)
_HALFB = pltpu.VMEM((CHUNKB, D), _f32)


def _sc_a(src, dst, b3e, t12):
    return pl.kernel(
        _sc_pass_a,
        out_type=[_sds((E, D)), _sds((NW, 2, D))],
        mesh=_mesh,
        scratch_types=[_IDX, _IDX, _ROWS, _ROWS, _HALF,
                       pltpu.VMEM((2, D), _f32),
                       pltpu.SemaphoreType.DMA],
    )(src, dst, b3e, t12)


def _sc_b(src, dst, etmp, ef, t23, st, zero):
    return pl.kernel(
        _sc_pass_b,
        out_type=[_sds((E, D)), _sds((NC, N, D2))],
        mesh=_mesh,
        scratch_types=[_IDXB, _IDXB, _IDX2DB, _ROWSB, _HALFB, _HALFB,
                       pltpu.VMEM((2, D), _f32),
                       pltpu.VMEM_SHARED((N, D2), _f32),
                       pltpu.SemaphoreType.DMA],
    )(src, dst, etmp, ef, t23, st, zero)


def _sc_c(src, dst, efn, t23, zero):
    return pl.kernel(
        _sc_pass_c,
        out_type=_sds((NC, N, D2)),
        mesh=_mesh,
        scratch_types=[_IDXB, _IDXB, _IDX2DB, _ROWSB, _HALFB,
                       pltpu.VMEM_SHARED((N, D2), _f32),
                       pltpu.SemaphoreType.DMA],
    )(src, dst, efn, t23, zero)


def _sc_g2(src, dst, t):
    return pl.kernel(
        _sc_gather2,
        out_type=_sds((E, D)),
        mesh=_mesh,
        scratch_types=[_IDX, _IDX, _ROWS, _ROWS, _HALF,
                       pltpu.SemaphoreType.DMA],
    )(src, dst, t)


def _sc_g2e(src, dst, t, ex):
    return pl.kernel(
        _sc_gather2e,
        out_type=_sds((E, D)),
        mesh=_mesh,
        scratch_types=[_IDX, _IDX, _ROWS, _ROWS, _HALF,
                       pltpu.SemaphoreType.DMA],
    )(src, dst, t, ex)


def _tc_node_mlp_body(x_ref, w1_ref, b1_ref, w2_ref, b2_ref, o_ref):
    hh = jnp.maximum(
        jnp.dot(x_ref[...], w1_ref[...],
                preferred_element_type=_f32) + b1_ref[...], 0.0)
    o_ref[...] = jnp.dot(hh, w2_ref[...],
                         preferred_element_type=_f32) + b2_ref[...]


def _tc_node_mlp(x, W1n, b1n, W2n, b2n):
    return pl.pallas_call(
        _tc_node_mlp_body,
        out_shape=_sds((N, D)),
    )(x, W1n, b1n[None], W2n, b2n[None])


def _tc_pair2_body(a_ref, b_ref, w_ref, bias_ref, o_ref):
    a, b = a_ref[...], b_ref[...]
    o_ref[:, 0:D] = (jnp.dot(a, w_ref[0], preferred_element_type=_f32)
                     + jnp.dot(b, w_ref[1], preferred_element_type=_f32)
                     + bias_ref[0, 0])
    o_ref[:, D:D2] = (jnp.dot(a, w_ref[2], preferred_element_type=_f32)
                      + jnp.dot(b, w_ref[3], preferred_element_type=_f32)
                      + bias_ref[0, 1])


def _tc_pair2(a, b, w4, bias2):
    return pl.pallas_call(
        _tc_pair2_body,
        out_shape=_sds((N, D2)),
    )(a, b, w4, bias2[None])


def _tc_mm_body(relu, x_ref, w_ref, b_ref, o_ref):
    y = jnp.dot(x_ref[...], w_ref[...],
                preferred_element_type=_f32) + b_ref[...]
    o_ref[...] = jnp.maximum(y, 0.0) if relu else y


def _tc_mm(x, w, b, relu, blk=8000):
    rows, din = x.shape
    dout = w.shape[1]
    return pl.pallas_call(
        functools.partial(_tc_mm_body, relu),
        grid=(rows // blk,),
        in_specs=[
            pl.BlockSpec((blk, din), lambda i: (i, 0)),
            pl.BlockSpec((din, dout), lambda i: (0, 0)),
            pl.BlockSpec((1, dout), lambda i: (0, 0)),
        ],
        out_specs=pl.BlockSpec((blk, dout), lambda i: (i, 0)),
        out_shape=_sds((rows, dout)),
    )(x, w, b[None])


def _tc_node5_body(h_ref, w_ref, b_ref, a1_ref, t12_ref, t23_ref):
    h = h_ref[...]
    mm = lambda i: jnp.dot(h, w_ref[i], preferred_element_type=_f32) \
        + b_ref[i, 0]
    a1_ref[...] = mm(0)
    t23_ref[:, 0:D] = mm(1)
    t23_ref[:, D:D2] = mm(2)
    t12_ref[:, 0:D] = mm(3)
    t12_ref[:, D:D2] = mm(4)


def _tc_node5(h, w5, b5):
    return pl.pallas_call(
        _tc_node5_body,
        out_shape=[_sds((N, D)), _sds((N, D2)), _sds((N, D2))],
    )(h, w5, b5)


def _tc_stats_body(s_ref, g_ref, b_ref, o_ref):
    tot = jnp.sum(s_ref[...], axis=0)
    mean = tot[0] / E
    var = tot[1] / E - mean * mean
    sv = g_ref[0] * lax.rsqrt(var + 1e-5)
    tv = b_ref[0] - mean * sv
    o_ref[...] = jnp.stack([sv, tv])


def _tc_stats(stats, g, b):
    return pl.pallas_call(
        _tc_stats_body,
        out_shape=_sds((2, D)),
    )(stats, g[None], b[None])


def _tc_hup_body(h_ref, a1_ref, nfd_ref, nbd_ref, g_ref, b_ref, o_ref):
    nf = nfd_ref[0, :, 0:D] + nfd_ref[1, :, 0:D]
    df = nfd_ref[0, :, D:D2] + nfd_ref[1, :, D:D2]
    nb = nbd_ref[0, :, 0:D] + nbd_ref[1, :, 0:D]
    db = nbd_ref[0, :, D:D2] + nbd_ref[1, :, D:D2]
    t = a1_ref[...] + nf / (df + 1e-6) + nb / (db + 1e-6)
    mu = jnp.mean(t, axis=0, keepdims=True)
    var = jnp.mean((t - mu) * (t - mu), axis=0, keepdims=True)
    bn = g_ref[...] * (t - mu) * lax.rsqrt(var + 1e-5) + b_ref[...]
    o_ref[...] = h_ref[...] + jnp.maximum(bn, 0.0)


def _tc_hup(h, a1, nfd, nbd, g, b):
    return pl.pallas_call(
        _tc_hup_body,
        out_shape=_sds((N, D)),
    )(h, a1, nfd, nbd, g[None], b[None])


_DBG_JAX_A = False
_DBG_JAX_B = False
_DBG_JAX_C = False
_DBG_JAX_G2 = False


def _jax_a(src, dst, b3e, t12):
    etmp = b3e + t12[:, 0:D][src] + t12[:, D:D2][dst]
    stats = jnp.stack([
        jnp.sum(etmp, axis=0), jnp.sum(etmp * etmp, axis=0)])[None]
    stats = jnp.concatenate([stats, jnp.zeros((NW - 1, 2, D), _f32)], 0)
    return etmp, stats


def _jax_b(src, dst, etmp, ef, t23, st, zero):
    efn = ef + jnp.maximum(etmp * st[0] + st[1], 0.0)
    sg = jax.nn.sigmoid(efn)
    nf = jax.ops.segment_sum(sg * t23[:, 0:D][src], dst, num_segments=N)
    df = jax.ops.segment_sum(sg, dst, num_segments=N)
    nfd = jnp.concatenate([nf, df], axis=1)[None]
    nfd = jnp.concatenate([nfd, jnp.zeros((1, N, D2), _f32)], 0)
    return efn, nfd


def _jax_c(src, dst, efn, t23, zero):
    sg = jax.nn.sigmoid(efn)
    nb = jax.ops.segment_sum(sg * t23[:, D:D2][dst], src, num_segments=N)
    db = jax.ops.segment_sum(sg, src, num_segments=N)
    nbd = jnp.concatenate([nb, db], axis=1)[None]
    return jnp.concatenate([nbd, jnp.zeros((1, N, D2), _f32)], 0)


def kernel(x, edge_index, e, W1n, b1n, W2n, b2n, W1e, b1e, W2e, b2e,
           gA1, bgA1, gA2, bgA2, gA3, bgA3, gB1, bgB1, gB2, bgB2, gB3, bgB3,
           gam_h, bet_h, gam_e, bet_e, pW1, pb1, pW2, pb2):
    src = edge_index[0]
    dst = edge_index[1]
    zero2 = jnp.zeros((N, D2), _f32)

    h = _tc_node_mlp(x, W1n, b1n, W2n, b2n)
    x2 = jax.random.normal(jax.random.key(1), (N, D), dtype=_f32)

    tuv = _tc_pair2(h, x2,
                    jnp.stack([W1e[0:64], W1e[128:192],
                               W1e[64:128], W1e[192:256]]),
                    jnp.stack([b1e, jnp.zeros((D,), _f32)]))
    he = _sc_g2(src, dst, tuv)
    ef = _tc_mm(he, W2e, b2e, relu=True)

    w5 = jnp.stack([gA1, gA2, gA3, gB1, gB2])
    b5 = jnp.stack([bgA1, bgA2, bgA3, bgB1, bgB2])[:, :, None, :]

    for l in range(L):
        a1t, t12, t23 = _tc_node5(h, w5[:, l], b5[:, l])
        b3e = _tc_mm(ef, gB3[l], bgB3[l], relu=False)
        etmp, stats = (_jax_a if _DBG_JAX_A else _sc_a)(src, dst, b3e, t12)
        st = _tc_stats(stats, gam_e[l], bet_e[l])
        efn, nfd = (_jax_b if _DBG_JAX_B else _sc_b)(
            src, dst, etmp, ef, t23, st, zero2)
        nbd = (_jax_c if _DBG_JAX_C else _sc_c)(src, dst, efn, t23, zero2)
        ef = efn
        h = _tc_hup(h, a1t, nfd, nbd, gam_h[l], bet_h[l])

    thp = _tc_pair2(h, h,
                    jnp.stack([pW1[0:64], jnp.zeros((D, D), _f32),
                               pW1[64:128], jnp.zeros((D, D), _f32)]),
                    jnp.stack([pb1, jnp.zeros((D,), _f32)]))
    efp3 = _tc_mm(ef, pW1[128:192], jnp.zeros((D,), _f32), relu=False)
    ph = _sc_g2e(src, dst, thp, efp3)
    scores = _tc_mm(ph, pW2, pb2, relu=False)
    return scores


# speedup vs baseline: 3.6581x; 1.0013x over previous
import functools

import jax
import jax.numpy as jnp
from jax import lax
from jax.experimental import pallas as pl
from jax.experimental.pallas import tpu as pltpu
from jax.experimental.pallas import tpu_sc as plsc

L = 8
N = 10000
E = 320000
D = 64
D2 = 128
NC = 2
NS = 16
NW = NC * NS
EPW = E // NW
SUB = 40
KSUB = 5
CHUNK = SUB * KSUB
NCHUNK = EPW // CHUNK
CHUNKB = 80
NCHUNKB = EPW // CHUNKB

_mesh = plsc.VectorSubcoreMesh(core_axis_name="c", subcore_axis_name="s")
_f32 = jnp.float32


def _wid():
    return lax.axis_index("s") * NC + lax.axis_index("c")


def _sc_pass_a(src_h, dst_h, b3e_h, t12_h, etmp_h, stats_h,
               idx1_v, idx2_v, r1_v, r2_v, acc_v, st_v, sem):
    wid = _wid()
    ebase = wid * EPW

    def chunk_body(ci, carry):
        off = ebase + ci * CHUNK
        pltpu.sync_copy(src_h.at[pl.ds(off, CHUNK)], idx1_v)
        pltpu.sync_copy(dst_h.at[pl.ds(off, CHUNK)], idx2_v)
        cps = []
        for k in range(KSUB):
            sl = pl.ds(SUB * k, SUB)
            cps.append(pltpu.async_copy(
                t12_h.at[idx1_v.at[sl]], r1_v.at[sl], sem))
            cps.append(pltpu.async_copy(
                t12_h.at[idx2_v.at[sl]], r2_v.at[sl], sem))
        pltpu.sync_copy(b3e_h.at[pl.ds(off, CHUNK)], acc_v)
        for cp in cps:
            cp.wait()

        @plsc.parallel_loop(0, CHUNK, unroll=4, carry=tuple(carry))
        def row_sums(r, c2):
            sums = list(c2)
            for j in range(4):
                sl = pl.ds(16 * j, 16)
                sh = pl.ds(64 + 16 * j, 16)
                a = acc_v[r, sl] + r1_v[r, sl] + r2_v[r, sh]
                acc_v[r, sl] = a
                sums[j] = sums[j] + a
                sums[4 + j] = sums[4 + j] + a * a
            return tuple(sums)

        carry = row_sums
        pltpu.sync_copy(acc_v, etmp_h.at[pl.ds(off, CHUNK)])
        return carry

    z = jnp.zeros((16,), _f32)
    sums = lax.fori_loop(0, NCHUNK, chunk_body, (z,) * 8)
    for j in range(4):
        st_v[0, pl.ds(16 * j, 16)] = sums[j]
        st_v[1, pl.ds(16 * j, 16)] = sums[4 + j]
    pltpu.sync_copy(st_v, stats_h.at[wid])


def _sc_pass_b(src_h, dst_h, etmp_h, ef_h, t23_h, st_h, zero_h,
               efn_h, nfd_h,
               idx1_v, idx2_v, idx2d_v, r1_v, et_v, ef_v, st_v, nfd_sh, sem):
    cid = lax.axis_index("c")
    sid = lax.axis_index("s")
    wid = sid * NC + cid
    ebase = wid * EPW

    @pl.when(sid == 0)
    def _():
        pltpu.sync_copy(zero_h, nfd_sh)
    plsc.subcore_barrier()

    pltpu.sync_copy(st_h, st_v)
    sv = [st_v[0, pl.ds(16 * j, 16)] for j in range(4)]
    tv = [st_v[1, pl.ds(16 * j, 16)] for j in range(4)]

    def chunk_body(ci, carry):
        off = ebase + ci * CHUNKB
        pltpu.sync_copy(src_h.at[pl.ds(off, CHUNKB)], idx1_v)
        pltpu.sync_copy(dst_h.at[pl.ds(off, CHUNKB)], idx2_v)
        cp = pltpu.async_copy(t23_h.at[idx1_v], r1_v, sem)
        pltpu.sync_copy(etmp_h.at[pl.ds(off, CHUNKB)], et_v)
        pltpu.sync_copy(ef_h.at[pl.ds(off, CHUNKB)], ef_v)
        cp.wait()

        @plsc.parallel_loop(0, CHUNKB, unroll=4)
        def _rowb(r):
            for j in range(4):
                sl = pl.ds(16 * j, 16)
                sh = pl.ds(64 + 16 * j, 16)
                x = jnp.maximum(et_v[r, sl] * sv[j] + tv[j], 0.0) + ef_v[r, sl]
                ef_v[r, sl] = x
                sg = 1.0 / (1.0 + jnp.exp(-x))
                r1_v[r, sl] = sg * r1_v[r, sl]
                r1_v[r, sh] = sg
        pltpu.sync_copy(ef_v, efn_h.at[pl.ds(off, CHUNKB)])
        for m in range(CHUNKB // 16):
            idx2d_v[0, pl.ds(16 * m, 16)] = idx2_v[pl.ds(16 * m, 16)]
        pltpu.sync_copy(r1_v, nfd_sh.at[idx2d_v.at[0]], add=True)
        return carry

    lax.fori_loop(0, NCHUNKB, chunk_body, 0)
    plsc.subcore_barrier()

    @pl.when(sid == 0)
    def _():
        pltpu.sync_copy(nfd_sh, nfd_h.at[cid])


def _sc_pass_c(src_h, dst_h, efn_h, t23_h, zero_h,
               nbd_h,
               idx1_v, idx2_v, idx2d_v, r1_v, ef_v, nbd_sh, sem):
    cid = lax.axis_index("c")
    sid = lax.axis_index("s")
    wid = sid * NC + cid
    ebase = wid * EPW

    @pl.when(sid == 0)
    def _():
        pltpu.sync_copy(zero_h, nbd_sh)
    plsc.subcore_barrier()

    def chunk_body(ci, carry):
        off = ebase + ci * CHUNKB
        pltpu.sync_copy(src_h.at[pl.ds(off, CHUNKB)], idx1_v)
        pltpu.sync_copy(dst_h.at[pl.ds(off, CHUNKB)], idx2_v)
        cp = pltpu.async_copy(t23_h.at[idx2_v], r1_v, sem)
        pltpu.sync_copy(efn_h.at[pl.ds(off, CHUNKB)], ef_v)
        cp.wait()

        @plsc.parallel_loop(0, CHUNKB, unroll=4)
        def _rowc(r):
            for j in range(4):
                sl = pl.ds(16 * j, 16)
                sh = pl.ds(64 + 16 * j, 16)
                sg = 1.0 / (1.0 + jnp.exp(-ef_v[r, sl]))
                r1_v[r, sl] = sg * r1_v[r, sh]
                r1_v[r, sh] = sg
        for m in range(CHUNKB // 16):
            idx2d_v[0, pl.ds(16 * m, 16)] = idx1_v[pl.ds(16 * m, 16)]
        pltpu.sync_copy(r1_v, nbd_sh.at[idx2d_v.at[0]], add=True)
        return carry

    lax.fori_loop(0, NCHUNKB, chunk_body, 0)
    plsc.subcore_barrier()

    @pl.when(sid == 0)
    def _():
        pltpu.sync_copy(nbd_sh, nbd_h.at[cid])


def _sc_gather2(src_h, dst_h, t_h, out_h,
                idx1_v, idx2_v, r1_v, r2_v, o_v, sem):
    wid = _wid()
    ebase = wid * EPW

    def chunk_body(ci, carry):
        off = ebase + ci * CHUNK
        pltpu.sync_copy(src_h.at[pl.ds(off, CHUNK)], idx1_v)
        pltpu.sync_copy(dst_h.at[pl.ds(off, CHUNK)], idx2_v)
        cps = []
        for k in range(KSUB):
            sl = pl.ds(SUB * k, SUB)
            cps.append(pltpu.async_copy(
                t_h.at[idx1_v.at[sl]], r1_v.at[sl], sem))
            cps.append(pltpu.async_copy(
                t_h.at[idx2_v.at[sl]], r2_v.at[sl], sem))
        for cp in cps:
            cp.wait()

        @plsc.parallel_loop(0, CHUNK, unroll=4)
        def _rowg(r):
            for j in range(4):
                sl = pl.ds(16 * j, 16)
                sh = pl.ds(64 + 16 * j, 16)
                o_v[r, sl] = jnp.maximum(r1_v[r, sl] + r2_v[r, sh], 0.0)
        pltpu.sync_copy(o_v, out_h.at[pl.ds(off, CHUNK)])
        return carry

    lax.fori_loop(0, NCHUNK, chunk_body, 0)


def _sc_gather2e(src_h, dst_h, t_h, ex_h, out_h,
                 idx1_v, idx2_v, r1_v, r2_v, ex_v, sem):
    wid = _wid()
    ebase = wid * EPW

    def chunk_body(ci, carry):
        off = ebase + ci * CHUNK
        pltpu.sync_copy(src_h.at[pl.ds(off, CHUNK)], idx1_v)
        pltpu.sync_copy(dst_h.at[pl.ds(off, CHUNK)], idx2_v)
        cps = []
        for k in range(KSUB):
            sl = pl.ds(SUB * k, SUB)
            cps.append(pltpu.async_copy(
                t_h.at[idx1_v.at[sl]], r1_v.at[sl], sem))
            cps.append(pltpu.async_copy(
                t_h.at[idx2_v.at[sl]], r2_v.at[sl], sem))
        pltpu.sync_copy(ex_h.at[pl.ds(off, CHUNK)], ex_v)
        for cp in cps:
            cp.wait()

        @plsc.parallel_loop(0, CHUNK, unroll=4)
        def _rowge(r):
            for j in range(4):
                sl = pl.ds(16 * j, 16)
                sh = pl.ds(64 + 16 * j, 16)
                ex_v[r, sl] = jnp.maximum(
                    r1_v[r, sl] + r2_v[r, sh] + ex_v[r, sl], 0.0)
        pltpu.sync_copy(ex_v, out_h.at[pl.ds(off, CHUNK)])
        return carry

    lax.fori_loop(0, NCHUNK, chunk_body, 0)


def _sds(shape):
    return jax.ShapeDtypeStruct(shape, _f32)


_IDX = pltpu.VMEM((CHUNK,), jnp.int32)
_ROWS = pltpu.VMEM((CHUNK, D2), _f32)
_HALF = pltpu.VMEM((CHUNK, D), _f32)
_IDXB = pltpu.VMEM((CHUNKB,), jnp.int32)
_IDX2DB = pltpu.VMEM((1, CHUNKB), jnp.int32)
_ROWSB = pltpu.VMEM((CHUNKB, D2), _f32)
_HALFB = pltpu.VMEM((CHUNKB, D), _f32)


def _sc_a(src, dst, b3e, t12):
    return pl.kernel(
        _sc_pass_a,
        out_type=[_sds((E, D)), _sds((NW, 2, D))],
        mesh=_mesh,
        scratch_types=[_IDX, _IDX, _ROWS, _ROWS, _HALF,
                       pltpu.VMEM((2, D), _f32),
                       pltpu.SemaphoreType.DMA],
    )(src, dst, b3e, t12)


def _sc_b(src, dst, etmp, ef, t23, st, zero):
    return pl.kernel(
        _sc_pass_b,
        out_type=[_sds((E, D)), _sds((NC, N, D2))],
        mesh=_mesh,
        scratch_types=[_IDXB, _IDXB, _IDX2DB, _ROWSB, _HALFB, _HALFB,
                       pltpu.VMEM((2, D), _f32),
                       pltpu.VMEM_SHARED((N, D2), _f32),
                       pltpu.SemaphoreType.DMA],
    )(src, dst, etmp, ef, t23, st, zero)


def _sc_c(src, dst, efn, t23, zero):
    return pl.kernel(
        _sc_pass_c,
        out_type=_sds((NC, N, D2)),
        mesh=_mesh,
        scratch_types=[_IDXB, _IDXB, _IDX2DB, _ROWSB, _HALFB,
                       pltpu.VMEM_SHARED((N, D2), _f32),
                       pltpu.SemaphoreType.DMA],
    )(src, dst, efn, t23, zero)


def _sc_g2(src, dst, t):
    return pl.kernel(
        _sc_gather2,
        out_type=_sds((E, D)),
        mesh=_mesh,
        scratch_types=[_IDX, _IDX, _ROWS, _ROWS, _HALF,
                       pltpu.SemaphoreType.DMA],
    )(src, dst, t)


def _sc_g2e(src, dst, t, ex):
    return pl.kernel(
        _sc_gather2e,
        out_type=_sds((E, D)),
        mesh=_mesh,
        scratch_types=[_IDX, _IDX, _ROWS, _ROWS, _HALF,
                       pltpu.SemaphoreType.DMA],
    )(src, dst, t, ex)


def _tc_node_mlp_body(x_ref, w1_ref, b1_ref, w2_ref, b2_ref, o_ref):
    hh = jnp.maximum(
        jnp.dot(x_ref[...], w1_ref[...],
                preferred_element_type=_f32) + b1_ref[...], 0.0)
    o_ref[...] = jnp.dot(hh, w2_ref[...],
                         preferred_element_type=_f32) + b2_ref[...]


def _tc_node_mlp(x, W1n, b1n, W2n, b2n):
    return pl.pallas_call(
        _tc_node_mlp_body,
        out_shape=_sds((N, D)),
    )(x, W1n, b1n[None], W2n, b2n[None])


def _tc_pair2_body(a_ref, b_ref, w_ref, bias_ref, o_ref):
    a, b = a_ref[...], b_ref[...]
    o_ref[:, 0:D] = (jnp.dot(a, w_ref[0], preferred_element_type=_f32)
                     + jnp.dot(b, w_ref[1], preferred_element_type=_f32)
                     + bias_ref[0, 0])
    o_ref[:, D:D2] = (jnp.dot(a, w_ref[2], preferred_element_type=_f32)
                      + jnp.dot(b, w_ref[3], preferred_element_type=_f32)
                      + bias_ref[0, 1])


def _tc_pair2(a, b, w4, bias2):
    return pl.pallas_call(
        _tc_pair2_body,
        out_shape=_sds((N, D2)),
    )(a, b, w4, bias2[None])


def _tc_mm_body(relu, x_ref, w_ref, b_ref, o_ref):
    y = jnp.dot(x_ref[...], w_ref[...],
                preferred_element_type=_f32) + b_ref[...]
    o_ref[...] = jnp.maximum(y, 0.0) if relu else y


def _tc_mm(x, w, b, relu, blk=8000):
    rows, din = x.shape
    dout = w.shape[1]
    return pl.pallas_call(
        functools.partial(_tc_mm_body, relu),
        grid=(rows // blk,),
        in_specs=[
            pl.BlockSpec((blk, din), lambda i: (i, 0)),
            pl.BlockSpec((din, dout), lambda i: (0, 0)),
            pl.BlockSpec((1, dout), lambda i: (0, 0)),
        ],
        out_specs=pl.BlockSpec((blk, dout), lambda i: (i, 0)),
        out_shape=_sds((rows, dout)),
    )(x, w, b[None])


def _tc_node5_body(h_ref, w_ref, b_ref, a1_ref, t12_ref, t23_ref):
    h = h_ref[...]
    mm = lambda i: jnp.dot(h, w_ref[i], preferred_element_type=_f32) \
        + b_ref[i, 0]
    a1_ref[...] = mm(0)
    t23_ref[:, 0:D] = mm(1)
    t23_ref[:, D:D2] = mm(2)
    t12_ref[:, 0:D] = mm(3)
    t12_ref[:, D:D2] = mm(4)


def _tc_node5(h, w5, b5):
    return pl.pallas_call(
        _tc_node5_body,
        out_shape=[_sds((N, D)), _sds((N, D2)), _sds((N, D2))],
    )(h, w5, b5)


def _tc_stats_body(s_ref, g_ref, b_ref, o_ref):
    tot = jnp.sum(s_ref[...], axis=0)
    mean = tot[0] / E
    var = tot[1] / E - mean * mean
    sv = g_ref[0] * lax.rsqrt(var + 1e-5)
    tv = b_ref[0] - mean * sv
    o_ref[...] = jnp.stack([sv, tv])


def _tc_stats(stats, g, b):
    return pl.pallas_call(
        _tc_stats_body,
        out_shape=_sds((2, D)),
    )(stats, g[None], b[None])


def _tc_hup_body(h_ref, a1_ref, nfd_ref, nbd_ref, g_ref, b_ref, o_ref):
    nf = nfd_ref[0, :, 0:D] + nfd_ref[1, :, 0:D]
    df = nfd_ref[0, :, D:D2] + nfd_ref[1, :, D:D2]
    nb = nbd_ref[0, :, 0:D] + nbd_ref[1, :, 0:D]
    db = nbd_ref[0, :, D:D2] + nbd_ref[1, :, D:D2]
    t = a1_ref[...] + nf / (df + 1e-6) + nb / (db + 1e-6)
    mu = jnp.mean(t, axis=0, keepdims=True)
    var = jnp.mean((t - mu) * (t - mu), axis=0, keepdims=True)
    bn = g_ref[...] * (t - mu) * lax.rsqrt(var + 1e-5) + b_ref[...]
    o_ref[...] = h_ref[...] + jnp.maximum(bn, 0.0)


def _tc_hup(h, a1, nfd, nbd, g, b):
    return pl.pallas_call(
        _tc_hup_body,
        out_shape=_sds((N, D)),
    )(h, a1, nfd, nbd, g[None], b[None])


_DBG_JAX_A = False
_DBG_JAX_B = False
_DBG_JAX_C = False
_DBG_JAX_G2 = False


def _jax_a(src, dst, b3e, t12):
    etmp = b3e + t12[:, 0:D][src] + t12[:, D:D2][dst]
    stats = jnp.stack([
        jnp.sum(etmp, axis=0), jnp.sum(etmp * etmp, axis=0)])[None]
    stats = jnp.concatenate([stats, jnp.zeros((NW - 1, 2, D), _f32)], 0)
    return etmp, stats


def _jax_b(src, dst, etmp, ef, t23, st, zero):
    efn = ef + jnp.maximum(etmp * st[0] + st[1], 0.0)
    sg = jax.nn.sigmoid(efn)
    nf = jax.ops.segment_sum(sg * t23[:, 0:D][src], dst, num_segments=N)
    df = jax.ops.segment_sum(sg, dst, num_segments=N)
    nfd = jnp.concatenate([nf, df], axis=1)[None]
    nfd = jnp.concatenate([nfd, jnp.zeros((1, N, D2), _f32)], 0)
    return efn, nfd


def _jax_c(src, dst, efn, t23, zero):
    sg = jax.nn.sigmoid(efn)
    nb = jax.ops.segment_sum(sg * t23[:, D:D2][dst], src, num_segments=N)
    db = jax.ops.segment_sum(sg, src, num_segments=N)
    nbd = jnp.concatenate([nb, db], axis=1)[None]
    return jnp.concatenate([nbd, jnp.zeros((1, N, D2), _f32)], 0)


def kernel(x, edge_index, e, W1n, b1n, W2n, b2n, W1e, b1e, W2e, b2e,
           gA1, bgA1, gA2, bgA2, gA3, bgA3, gB1, bgB1, gB2, bgB2, gB3, bgB3,
           gam_h, bet_h, gam_e, bet_e, pW1, pb1, pW2, pb2):
    src = edge_index[0]
    dst = edge_index[1]
    zero2 = jnp.zeros((N, D2), _f32)

    h = _tc_node_mlp(x, W1n, b1n, W2n, b2n)
    x2 = jax.random.normal(jax.random.key(1), (N, D), dtype=_f32)

    tuv = _tc_pair2(h, x2,
                    jnp.stack([W1e[0:64], W1e[128:192],
                               W1e[64:128], W1e[192:256]]),
                    jnp.stack([b1e, jnp.zeros((D,), _f32)]))
    he = _sc_g2(src, dst, tuv)
    ef = _tc_mm(he, W2e, b2e, relu=True)

    w5 = jnp.stack([gA1, gA2, gA3, gB1, gB2])
    b5 = jnp.stack([bgA1, bgA2, bgA3, bgB1, bgB2])[:, :, None, :]

    for l in range(L):
        a1t, t12, t23 = _tc_node5(h, w5[:, l], b5[:, l])
        b3e = _tc_mm(ef, gB3[l], bgB3[l], relu=False)
        etmp, stats = (_jax_a if _DBG_JAX_A else _sc_a)(src, dst, b3e, t12)
        st = _tc_stats(stats, gam_e[l], bet_e[l])
        efn, nfd = (_jax_b if _DBG_JAX_B else _sc_b)(
            src, dst, etmp, ef, t23, st, zero2)
        nbd = (_jax_c if _DBG_JAX_C else _sc_c)(src, dst, efn, t23, zero2)
        ef = efn
        h = _tc_hup(h, a1t, nfd, nbd, gam_h[l], bet_h[l])

    thp = _tc_pair2(h, h,
                    jnp.stack([pW1[0:64], jnp.zeros((D, D), _f32),
                               pW1[64:128], jnp.zeros((D, D), _f32)]),
                    jnp.stack([pb1, jnp.zeros((D,), _f32)]))
    efp3 = _tc_mm(ef, pW1[128:192], jnp.zeros((D,), _f32), relu=False)
    ph = _sc_g2e(src, dst, thp, efp3)
    scores = _tc_mm(ph, pW2, pb2, relu=False)
    return scores


# speedup vs baseline: 4.5988x; 1.2572x over previous
import functools

import jax
import jax.numpy as jnp
from jax import lax
from jax.experimental import pallas as pl
from jax.experimental.pallas import tpu as pltpu
from jax.experimental.pallas import tpu_sc as plsc

L = 8
N = 10000
E = 320000
D = 64
D2 = 128
NC = 2
NS = 16
NW = NC * NS
EPW = E // NW
SUB = 40
KSUB = 5
CHUNK = SUB * KSUB
NCHUNK = EPW // CHUNK
CHUNKB = 80
NCHUNKB = EPW // CHUNKB

_mesh = plsc.VectorSubcoreMesh(core_axis_name="c", subcore_axis_name="s")
_f32 = jnp.float32


def _wid():
    return lax.axis_index("s") * NC + lax.axis_index("c")


def _sc_pass_a(src_h, dst_h, b3e_h, t12_h, etmp_h, stats_h,
               idx1_v, idx2_v, r1_v, r2_v, acc_v, st_v, sem, semi, semx):
    wid = _wid()
    ebase = wid * EPW

    def chunk_body(ci, carry):
        off = ebase + ci * CHUNK
        ixs = [pltpu.async_copy(src_h.at[pl.ds(off, CHUNK)], idx1_v, semx),
               pltpu.async_copy(dst_h.at[pl.ds(off, CHUNK)], idx2_v, semx)]
        ins = [pltpu.async_copy(b3e_h.at[pl.ds(off, CHUNK)], acc_v, semi)]
        ixs[0].wait()
        ixs[1].wait()
        cps = []
        for k in range(KSUB):
            sl = pl.ds(SUB * k, SUB)
            cps.append(pltpu.async_copy(
                t12_h.at[idx1_v.at[sl]], r1_v.at[sl], sem))
            cps.append(pltpu.async_copy(
                t12_h.at[idx2_v.at[sl]], r2_v.at[sl], sem))
        ins[0].wait()
        for cp in cps:
            cp.wait()

        @plsc.parallel_loop(0, CHUNK, unroll=4, carry=tuple(carry))
        def row_sums(r, c2):
            sums = list(c2)
            for j in range(4):
                sl = pl.ds(16 * j, 16)
                sh = pl.ds(64 + 16 * j, 16)
                a = acc_v[r, sl] + r1_v[r, sl] + r2_v[r, sh]
                acc_v[r, sl] = a
                sums[j] = sums[j] + a
                sums[4 + j] = sums[4 + j] + a * a
            return tuple(sums)

        carry = row_sums
        pltpu.sync_copy(acc_v, etmp_h.at[pl.ds(off, CHUNK)])
        return carry

    z = jnp.zeros((16,), _f32)
    sums = lax.fori_loop(0, NCHUNK, chunk_body, (z,) * 8)
    for j in range(4):
        st_v[0, pl.ds(16 * j, 16)] = sums[j]
        st_v[1, pl.ds(16 * j, 16)] = sums[4 + j]
    pltpu.sync_copy(st_v, stats_h.at[wid])


def _sc_pass_b(src_h, dst_h, etmp_h, ef_h, t23_h, st_h, zero_h,
               efn_h, nfd_h,
               idx1_v, idx2_v, idx2d_v, r1_v, et_v, ef_v, st_v, nfd_sh, sem, semi, semx):
    cid = lax.axis_index("c")
    sid = lax.axis_index("s")
    wid = sid * NC + cid
    ebase = wid * EPW

    @pl.when(sid == 0)
    def _():
        pltpu.sync_copy(zero_h, nfd_sh)
    plsc.subcore_barrier()

    pltpu.sync_copy(st_h, st_v)
    sv = [st_v[0, pl.ds(16 * j, 16)] for j in range(4)]
    tv = [st_v[1, pl.ds(16 * j, 16)] for j in range(4)]

    def chunk_body(ci, carry):
        off = ebase + ci * CHUNKB
        ix = pltpu.async_copy(src_h.at[pl.ds(off, CHUNKB)], idx1_v, semx)
        ins = [pltpu.async_copy(dst_h.at[pl.ds(off, CHUNKB)], idx2_v, semi),
               pltpu.async_copy(etmp_h.at[pl.ds(off, CHUNKB)], et_v, semi),
               pltpu.async_copy(ef_h.at[pl.ds(off, CHUNKB)], ef_v, semi)]
        ix.wait()
        cp = pltpu.async_copy(t23_h.at[idx1_v], r1_v, sem)
        ins[0].wait()
        ins[1].wait()
        ins[2].wait()
        cp.wait()

        @plsc.parallel_loop(0, CHUNKB, unroll=4)
        def _rowb(r):
            for j in range(4):
                sl = pl.ds(16 * j, 16)
                sh = pl.ds(64 + 16 * j, 16)
                x = jnp.maximum(et_v[r, sl] * sv[j] + tv[j], 0.0) + ef_v[r, sl]
                ef_v[r, sl] = x
                sg = 1.0 / (1.0 + jnp.exp(-x))
                r1_v[r, sl] = sg * r1_v[r, sl]
                r1_v[r, sh] = sg
        pltpu.sync_copy(ef_v, efn_h.at[pl.ds(off, CHUNKB)])
        for m in range(CHUNKB // 16):
            idx2d_v[0, pl.ds(16 * m, 16)] = idx2_v[pl.ds(16 * m, 16)]
        pltpu.sync_copy(r1_v, nfd_sh.at[idx2d_v.at[0]], add=True)
        return carry

    lax.fori_loop(0, NCHUNKB, chunk_body, 0)
    plsc.subcore_barrier()

    @pl.when(sid == 0)
    def _():
        pltpu.sync_copy(nfd_sh, nfd_h.at[cid])


def _sc_pass_c(src_h, dst_h, efn_h, t23_h, zero_h,
               nbd_h,
               idx1_v, idx2_v, idx2d_v, r1_v, ef_v, nbd_sh, sem, semi, semx):
    cid = lax.axis_index("c")
    sid = lax.axis_index("s")
    wid = sid * NC + cid
    ebase = wid * EPW

    @pl.when(sid == 0)
    def _():
        pltpu.sync_copy(zero_h, nbd_sh)
    plsc.subcore_barrier()

    def chunk_body(ci, carry):
        off = ebase + ci * CHUNKB
        ix = pltpu.async_copy(dst_h.at[pl.ds(off, CHUNKB)], idx2_v, semx)
        ins = [pltpu.async_copy(src_h.at[pl.ds(off, CHUNKB)], idx1_v, semi),
               pltpu.async_copy(efn_h.at[pl.ds(off, CHUNKB)], ef_v, semi)]
        ix.wait()
        cp = pltpu.async_copy(t23_h.at[idx2_v], r1_v, sem)
        ins[0].wait()
        ins[1].wait()
        cp.wait()

        @plsc.parallel_loop(0, CHUNKB, unroll=4)
        def _rowc(r):
            for j in range(4):
                sl = pl.ds(16 * j, 16)
                sh = pl.ds(64 + 16 * j, 16)
                sg = 1.0 / (1.0 + jnp.exp(-ef_v[r, sl]))
                r1_v[r, sl] = sg * r1_v[r, sh]
                r1_v[r, sh] = sg
        for m in range(CHUNKB // 16):
            idx2d_v[0, pl.ds(16 * m, 16)] = idx1_v[pl.ds(16 * m, 16)]
        pltpu.sync_copy(r1_v, nbd_sh.at[idx2d_v.at[0]], add=True)
        return carry

    lax.fori_loop(0, NCHUNKB, chunk_body, 0)
    plsc.subcore_barrier()

    @pl.when(sid == 0)
    def _():
        pltpu.sync_copy(nbd_sh, nbd_h.at[cid])


def _sc_gather2(src_h, dst_h, t_h, out_h,
                idx1_v, idx2_v, r1_v, r2_v, o_v, sem, semi, semx):
    wid = _wid()
    ebase = wid * EPW

    def chunk_body(ci, carry):
        off = ebase + ci * CHUNK
        ixs = [pltpu.async_copy(src_h.at[pl.ds(off, CHUNK)], idx1_v, semx),
               pltpu.async_copy(dst_h.at[pl.ds(off, CHUNK)], idx2_v, semx)]
        ixs[0].wait()
        ixs[1].wait()
        cps = []
        for k in range(KSUB):
            sl = pl.ds(SUB * k, SUB)
            cps.append(pltpu.async_copy(
                t_h.at[idx1_v.at[sl]], r1_v.at[sl], sem))
            cps.append(pltpu.async_copy(
                t_h.at[idx2_v.at[sl]], r2_v.at[sl], sem))
        for cp in cps:
            cp.wait()

        @plsc.parallel_loop(0, CHUNK, unroll=4)
        def _rowg(r):
            for j in range(4):
                sl = pl.ds(16 * j, 16)
                sh = pl.ds(64 + 16 * j, 16)
                o_v[r, sl] = jnp.maximum(r1_v[r, sl] + r2_v[r, sh], 0.0)
        pltpu.sync_copy(o_v, out_h.at[pl.ds(off, CHUNK)])
        return carry

    lax.fori_loop(0, NCHUNK, chunk_body, 0)


def _sc_gather2e(src_h, dst_h, t_h, ex_h, out_h,
                 idx1_v, idx2_v, r1_v, r2_v, ex_v, sem, semi, semx):
    wid = _wid()
    ebase = wid * EPW

    def chunk_body(ci, carry):
        off = ebase + ci * CHUNK
        ixs = [pltpu.async_copy(src_h.at[pl.ds(off, CHUNK)], idx1_v, semx),
               pltpu.async_copy(dst_h.at[pl.ds(off, CHUNK)], idx2_v, semx)]
        ins = [pltpu.async_copy(ex_h.at[pl.ds(off, CHUNK)], ex_v, semi)]
        ixs[0].wait()
        ixs[1].wait()
        cps = []
        for k in range(KSUB):
            sl = pl.ds(SUB * k, SUB)
            cps.append(pltpu.async_copy(
                t_h.at[idx1_v.at[sl]], r1_v.at[sl], sem))
            cps.append(pltpu.async_copy(
                t_h.at[idx2_v.at[sl]], r2_v.at[sl], sem))
        ins[0].wait()
        for cp in cps:
            cp.wait()

        @plsc.parallel_loop(0, CHUNK, unroll=4)
        def _rowge(r):
            for j in range(4):
                sl = pl.ds(16 * j, 16)
                sh = pl.ds(64 + 16 * j, 16)
                ex_v[r, sl] = jnp.maximum(
                    r1_v[r, sl] + r2_v[r, sh] + ex_v[r, sl], 0.0)
        pltpu.sync_copy(ex_v, out_h.at[pl.ds(off, CHUNK)])
        return carry

    lax.fori_loop(0, NCHUNK, chunk_body, 0)


def _sds(shape):
    return jax.ShapeDtypeStruct(shape, _f32)


_IDX = pltpu.VMEM((CHUNK,), jnp.int32)
_ROWS = pltpu.VMEM((CHUNK, D2), _f32)
_HALF = pltpu.VMEM((CHUNK, D), _f32)
_IDXB = pltpu.VMEM((CHUNKB,), jnp.int32)
_IDX2DB = pltpu.VMEM((1, CHUNKB), jnp.int32)
_ROWSB = pltpu.VMEM((CHUNKB, D2), _f32)
_HALFB = pltpu.VMEM((CHUNKB, D), _f32)


def _sc_a(src, dst, b3e, t12):
    return pl.kernel(
        _sc_pass_a,
        out_type=[_sds((E, D)), _sds((NW, 2, D))],
        mesh=_mesh,
        scratch_types=[_IDX, _IDX, _ROWS, _ROWS, _HALF,
                       pltpu.VMEM((2, D), _f32),
                       pltpu.SemaphoreType.DMA,
                       pltpu.SemaphoreType.DMA,
                       pltpu.SemaphoreType.DMA],
    )(src, dst, b3e, t12)


def _sc_b(src, dst, etmp, ef, t23, st, zero):
    return pl.kernel(
        _sc_pass_b,
        out_type=[_sds((E, D)), _sds((NC, N, D2))],
        mesh=_mesh,
        scratch_types=[_IDXB, _IDXB, _IDX2DB, _ROWSB, _HALFB, _HALFB,
                       pltpu.VMEM((2, D), _f32),
                       pltpu.VMEM_SHARED((N, D2), _f32),
                       pltpu.SemaphoreType.DMA,
                       pltpu.SemaphoreType.DMA,
                       pltpu.SemaphoreType.DMA],
    )(src, dst, etmp, ef, t23, st, zero)


def _sc_c(src, dst, efn, t23, zero):
    return pl.kernel(
        _sc_pass_c,
        out_type=_sds((NC, N, D2)),
        mesh=_mesh,
        scratch_types=[_IDXB, _IDXB, _IDX2DB, _ROWSB, _HALFB,
                       pltpu.VMEM_SHARED((N, D2), _f32),
                       pltpu.SemaphoreType.DMA,
                       pltpu.SemaphoreType.DMA,
                       pltpu.SemaphoreType.DMA],
    )(src, dst, efn, t23, zero)


def _sc_g2(src, dst, t):
    return pl.kernel(
        _sc_gather2,
        out_type=_sds((E, D)),
        mesh=_mesh,
        scratch_types=[_IDX, _IDX, _ROWS, _ROWS, _HALF,
                       pltpu.SemaphoreType.DMA,
                       pltpu.SemaphoreType.DMA,
                       pltpu.SemaphoreType.DMA],
    )(src, dst, t)


def _sc_g2e(src, dst, t, ex):
    return pl.kernel(
        _sc_gather2e,
        out_type=_sds((E, D)),
        mesh=_mesh,
        scratch_types=[_IDX, _IDX, _ROWS, _ROWS, _HALF,
                       pltpu.SemaphoreType.DMA,
                       pltpu.SemaphoreType.DMA,
                       pltpu.SemaphoreType.DMA],
    )(src, dst, t, ex)


def _tc_node_mlp_body(x_ref, w1_ref, b1_ref, w2_ref, b2_ref, o_ref):
    hh = jnp.maximum(
        jnp.dot(x_ref[...], w1_ref[...],
                preferred_element_type=_f32) + b1_ref[...], 0.0)
    o_ref[...] = jnp.dot(hh, w2_ref[...],
                         preferred_element_type=_f32) + b2_ref[...]


def _tc_node_mlp(x, W1n, b1n, W2n, b2n):
    return pl.pallas_call(
        _tc_node_mlp_body,
        out_shape=_sds((N, D)),
    )(x, W1n, b1n[None], W2n, b2n[None])


def _tc_pair2_body(a_ref, b_ref, w_ref, bias_ref, o_ref):
    a, b = a_ref[...], b_ref[...]
    o_ref[:, 0:D] = (jnp.dot(a, w_ref[0], preferred_element_type=_f32)
                     + jnp.dot(b, w_ref[1], preferred_element_type=_f32)
                     + bias_ref[0, 0])
    o_ref[:, D:D2] = (jnp.dot(a, w_ref[2], preferred_element_type=_f32)
                      + jnp.dot(b, w_ref[3], preferred_element_type=_f32)
                      + bias_ref[0, 1])


def _tc_pair2(a, b, w4, bias2):
    return pl.pallas_call(
        _tc_pair2_body,
        out_shape=_sds((N, D2)),
    )(a, b, w4, bias2[None])


def _tc_mm_body(relu, x_ref, w_ref, b_ref, o_ref):
    y = jnp.dot(x_ref[...], w_ref[...],
                preferred_element_type=_f32) + b_ref[...]
    o_ref[...] = jnp.maximum(y, 0.0) if relu else y


def _tc_mm(x, w, b, relu, blk=8000):
    rows, din = x.shape
    dout = w.shape[1]
    return pl.pallas_call(
        functools.partial(_tc_mm_body, relu),
        grid=(rows // blk,),
        in_specs=[
            pl.BlockSpec((blk, din), lambda i: (i, 0)),
            pl.BlockSpec((din, dout), lambda i: (0, 0)),
            pl.BlockSpec((1, dout), lambda i: (0, 0)),
        ],
        out_specs=pl.BlockSpec((blk, dout), lambda i: (i, 0)),
        out_shape=_sds((rows, dout)),
    )(x, w, b[None])


def _tc_node5_body(h_ref, w_ref, b_ref, a1_ref, t12_ref, t23_ref):
    h = h_ref[...]
    mm = lambda i: jnp.dot(h, w_ref[i], preferred_element_type=_f32) \
        + b_ref[i, 0]
    a1_ref[...] = mm(0)
    t23_ref[:, 0:D] = mm(1)
    t23_ref[:, D:D2] = mm(2)
    t12_ref[:, 0:D] = mm(3)
    t12_ref[:, D:D2] = mm(4)


def _tc_node5(h, w5, b5):
    return pl.pallas_call(
        _tc_node5_body,
        out_shape=[_sds((N, D)), _sds((N, D2)), _sds((N, D2))],
    )(h, w5, b5)


def _tc_stats_body(s_ref, g_ref, b_ref, o_ref):
    tot = jnp.sum(s_ref[...], axis=0)
    mean = tot[0] / E
    var = tot[1] / E - mean * mean
    sv = g_ref[0] * lax.rsqrt(var + 1e-5)
    tv = b_ref[0] - mean * sv
    o_ref[...] = jnp.stack([sv, tv])


def _tc_stats(stats, g, b):
    return pl.pallas_call(
        _tc_stats_body,
        out_shape=_sds((2, D)),
    )(stats, g[None], b[None])


def _tc_hup_body(h_ref, a1_ref, nfd_ref, nbd_ref, g_ref, b_ref, o_ref):
    nf = nfd_ref[0, :, 0:D] + nfd_ref[1, :, 0:D]
    df = nfd_ref[0, :, D:D2] + nfd_ref[1, :, D:D2]
    nb = nbd_ref[0, :, 0:D] + nbd_ref[1, :, 0:D]
    db = nbd_ref[0, :, D:D2] + nbd_ref[1, :, D:D2]
    t = a1_ref[...] + nf / (df + 1e-6) + nb / (db + 1e-6)
    mu = jnp.mean(t, axis=0, keepdims=True)
    var = jnp.mean((t - mu) * (t - mu), axis=0, keepdims=True)
    bn = g_ref[...] * (t - mu) * lax.rsqrt(var + 1e-5) + b_ref[...]
    o_ref[...] = h_ref[...] + jnp.maximum(bn, 0.0)


def _tc_hup(h, a1, nfd, nbd, g, b):
    return pl.pallas_call(
        _tc_hup_body,
        out_shape=_sds((N, D)),
    )(h, a1, nfd, nbd, g[None], b[None])


_DBG_JAX_A = False
_DBG_JAX_B = False
_DBG_JAX_C = False
_DBG_JAX_G2 = False


def _jax_a(src, dst, b3e, t12):
    etmp = b3e + t12[:, 0:D][src] + t12[:, D:D2][dst]
    stats = jnp.stack([
        jnp.sum(etmp, axis=0), jnp.sum(etmp * etmp, axis=0)])[None]
    stats = jnp.concatenate([stats, jnp.zeros((NW - 1, 2, D), _f32)], 0)
    return etmp, stats


def _jax_b(src, dst, etmp, ef, t23, st, zero):
    efn = ef + jnp.maximum(etmp * st[0] + st[1], 0.0)
    sg = jax.nn.sigmoid(efn)
    nf = jax.ops.segment_sum(sg * t23[:, 0:D][src], dst, num_segments=N)
    df = jax.ops.segment_sum(sg, dst, num_segments=N)
    nfd = jnp.concatenate([nf, df], axis=1)[None]
    nfd = jnp.concatenate([nfd, jnp.zeros((1, N, D2), _f32)], 0)
    return efn, nfd


def _jax_c(src, dst, efn, t23, zero):
    sg = jax.nn.sigmoid(efn)
    nb = jax.ops.segment_sum(sg * t23[:, D:D2][dst], src, num_segments=N)
    db = jax.ops.segment_sum(sg, src, num_segments=N)
    nbd = jnp.concatenate([nb, db], axis=1)[None]
    return jnp.concatenate([nbd, jnp.zeros((1, N, D2), _f32)], 0)


def kernel(x, edge_index, e, W1n, b1n, W2n, b2n, W1e, b1e, W2e, b2e,
           gA1, bgA1, gA2, bgA2, gA3, bgA3, gB1, bgB1, gB2, bgB2, gB3, bgB3,
           gam_h, bet_h, gam_e, bet_e, pW1, pb1, pW2, pb2):
    src = edge_index[0]
    dst = edge_index[1]
    zero2 = jnp.zeros((N, D2), _f32)

    h = _tc_node_mlp(x, W1n, b1n, W2n, b2n)
    x2 = jax.random.normal(jax.random.key(1), (N, D), dtype=_f32)

    tuv = _tc_pair2(h, x2,
                    jnp.stack([W1e[0:64], W1e[128:192],
                               W1e[64:128], W1e[192:256]]),
                    jnp.stack([b1e, jnp.zeros((D,), _f32)]))
    he = _sc_g2(src, dst, tuv)
    ef = _tc_mm(he, W2e, b2e, relu=True)

    w5 = jnp.stack([gA1, gA2, gA3, gB1, gB2])
    b5 = jnp.stack([bgA1, bgA2, bgA3, bgB1, bgB2])[:, :, None, :]

    for l in range(L):
        a1t, t12, t23 = _tc_node5(h, w5[:, l], b5[:, l])
        b3e = _tc_mm(ef, gB3[l], bgB3[l], relu=False)
        etmp, stats = (_jax_a if _DBG_JAX_A else _sc_a)(src, dst, b3e, t12)
        st = _tc_stats(stats, gam_e[l], bet_e[l])
        efn, nfd = (_jax_b if _DBG_JAX_B else _sc_b)(
            src, dst, etmp, ef, t23, st, zero2)
        nbd = (_jax_c if _DBG_JAX_C else _sc_c)(src, dst, efn, t23, zero2)
        ef = efn
        h = _tc_hup(h, a1t, nfd, nbd, gam_h[l], bet_h[l])

    thp = _tc_pair2(h, h,
                    jnp.stack([pW1[0:64], jnp.zeros((D, D), _f32),
                               pW1[64:128], jnp.zeros((D, D), _f32)]),
                    jnp.stack([pb1, jnp.zeros((D,), _f32)]))
    efp3 = _tc_mm(ef, pW1[128:192], jnp.zeros((D,), _f32), relu=False)
    ph = _sc_g2e(src, dst, thp, efp3)
    scores = _tc_mm(ph, pW2, pb2, relu=False)
    return scores


# speedup vs baseline: 4.6576x; 1.0128x over previous
import functools

import jax
import jax.numpy as jnp
from jax import lax
from jax.experimental import pallas as pl
from jax.experimental.pallas import tpu as pltpu
from jax.experimental.pallas import tpu_sc as plsc

L = 8
N = 10000
E = 320000
D = 64
D2 = 128
NC = 2
NS = 16
NW = NC * NS
EPW = E // NW
SUB = 40
KSUB = 5
CHUNK = SUB * KSUB
NCHUNK = EPW // CHUNK
CHUNKB = 80
NCHUNKB = EPW // CHUNKB

_mesh = plsc.VectorSubcoreMesh(core_axis_name="c", subcore_axis_name="s")
_f32 = jnp.float32


def _wid():
    return lax.axis_index("s") * NC + lax.axis_index("c")


def _sc_pass_a(src_h, dst_h, b3e_h, t12_h, etmp_h, stats_h,
               idx1_v, idx2_v, r1_v, r2_v, acc_v, st_v, sem, semi, semx):
    wid = _wid()
    ebase = wid * EPW

    def chunk_body(ci, carry):
        off = ebase + ci * CHUNK
        ixs = [pltpu.async_copy(src_h.at[pl.ds(off, CHUNK)], idx1_v, semx),
               pltpu.async_copy(dst_h.at[pl.ds(off, CHUNK)], idx2_v, semx)]
        ins = [pltpu.async_copy(b3e_h.at[pl.ds(off, CHUNK)], acc_v, semi)]
        ixs[0].wait()
        ixs[1].wait()
        cps = []
        for k in range(KSUB):
            sl = pl.ds(SUB * k, SUB)
            cps.append(pltpu.async_copy(
                t12_h.at[idx1_v.at[sl]], r1_v.at[sl], sem))
            cps.append(pltpu.async_copy(
                t12_h.at[idx2_v.at[sl]], r2_v.at[sl], sem))
        ins[0].wait()
        for cp in cps:
            cp.wait()

        @plsc.parallel_loop(0, CHUNK, unroll=4, carry=tuple(carry))
        def row_sums(r, c2):
            sums = list(c2)
            for j in range(4):
                sl = pl.ds(16 * j, 16)
                sh = pl.ds(64 + 16 * j, 16)
                a = acc_v[r, sl] + r1_v[r, sl] + r2_v[r, sh]
                acc_v[r, sl] = a
                sums[j] = sums[j] + a
                sums[4 + j] = sums[4 + j] + a * a
            return tuple(sums)

        carry = row_sums
        pltpu.sync_copy(acc_v, etmp_h.at[pl.ds(off, CHUNK)])
        return carry

    z = jnp.zeros((16,), _f32)
    sums = lax.fori_loop(0, NCHUNK, chunk_body, (z,) * 8)
    for j in range(4):
        st_v[0, pl.ds(16 * j, 16)] = sums[j]
        st_v[1, pl.ds(16 * j, 16)] = sums[4 + j]
    pltpu.sync_copy(st_v, stats_h.at[wid])


def _sc_pass_b(src_h, dst_h, etmp_h, ef_h, t23_h, st_h, zero_h,
               efn_h, nfd_h,
               idx1_v, idx2_v, idx2d_v, r1_v, et_v, ef_v, st_v, nfd_sh, sem, semi, semx0, semx1):
    cid = lax.axis_index("c")
    sid = lax.axis_index("s")
    wid = sid * NC + cid
    ebase = wid * EPW

    @pl.when(sid == 0)
    def _():
        pltpu.sync_copy(zero_h, nfd_sh)
    plsc.subcore_barrier()

    pltpu.sync_copy(st_h, st_v)
    sv = [st_v[0, pl.ds(16 * j, 16)] for j in range(4)]
    tv = [st_v[1, pl.ds(16 * j, 16)] for j in range(4)]

    def fire_idx(ci, b):
        off2 = ebase + ci * CHUNKB
        sx = semx0 if b == 0 else semx1
        pltpu.async_copy(src_h.at[pl.ds(off2, CHUNKB)], idx1_v.at[b], sx)
        pltpu.async_copy(dst_h.at[pl.ds(off2, CHUNKB)], idx2_v.at[b], sx)

    def wait_idx(ci, b):
        off2 = ebase + ci * CHUNKB
        sx = semx0 if b == 0 else semx1
        pltpu.make_async_copy(
            src_h.at[pl.ds(off2, CHUNKB)], idx1_v.at[b], sx).wait()
        pltpu.make_async_copy(
            dst_h.at[pl.ds(off2, CHUNKB)], idx2_v.at[b], sx).wait()

    def do_chunk(ci, b, prefetch):
        off = ebase + ci * CHUNKB
        ins = [pltpu.async_copy(etmp_h.at[pl.ds(off, CHUNKB)], et_v, semi),
               pltpu.async_copy(ef_h.at[pl.ds(off, CHUNKB)], ef_v, semi)]
        if prefetch:
            fire_idx(ci + 1, 1 - b)
        wait_idx(ci, b)
        cp = pltpu.async_copy(t23_h.at[idx1_v.at[b]], r1_v, sem)
        ins[0].wait()
        ins[1].wait()
        cp.wait()

        @plsc.parallel_loop(0, CHUNKB, unroll=4)
        def _rowb(r):
            for j in range(4):
                sl = pl.ds(16 * j, 16)
                sh = pl.ds(64 + 16 * j, 16)
                x = jnp.maximum(et_v[r, sl] * sv[j] + tv[j], 0.0) + ef_v[r, sl]
                ef_v[r, sl] = x
                sg = 1.0 / (1.0 + jnp.exp(-x))
                r1_v[r, sl] = sg * r1_v[r, sl]
                r1_v[r, sh] = sg
        pltpu.sync_copy(ef_v, efn_h.at[pl.ds(off, CHUNKB)])
        for m in range(CHUNKB // 16):
            idx2d_v[0, pl.ds(16 * m, 16)] = idx2_v[b, pl.ds(16 * m, 16)]
        pltpu.sync_copy(r1_v, nfd_sh.at[idx2d_v.at[0]], add=True)

    fire_idx(0, 0)

    def pair_body(i, carry):
        do_chunk(2 * i, 0, True)
        do_chunk(2 * i + 1, 1, True)
        return carry

    lax.fori_loop(0, NCHUNKB // 2, pair_body, 0)
    do_chunk(NCHUNKB - 1, 0, False)
    plsc.subcore_barrier()

    @pl.when(sid == 0)
    def _():
        pltpu.sync_copy(nfd_sh, nfd_h.at[cid])


def _sc_pass_c(src_h, dst_h, efn_h, t23_h, zero_h,
               nbd_h,
               idx1_v, idx2_v, idx2d_v, r1_v, ef_v, nbd_sh, sem, semi, semx0, semx1):
    cid = lax.axis_index("c")
    sid = lax.axis_index("s")
    wid = sid * NC + cid
    ebase = wid * EPW

    @pl.when(sid == 0)
    def _():
        pltpu.sync_copy(zero_h, nbd_sh)
    plsc.subcore_barrier()

    def fire_idx(ci, b):
        off2 = ebase + ci * CHUNKB
        sx = semx0 if b == 0 else semx1
        pltpu.async_copy(src_h.at[pl.ds(off2, CHUNKB)], idx1_v.at[b], sx)
        pltpu.async_copy(dst_h.at[pl.ds(off2, CHUNKB)], idx2_v.at[b], sx)

    def wait_idx(ci, b):
        off2 = ebase + ci * CHUNKB
        sx = semx0 if b == 0 else semx1
        pltpu.make_async_copy(
            src_h.at[pl.ds(off2, CHUNKB)], idx1_v.at[b], sx).wait()
        pltpu.make_async_copy(
            dst_h.at[pl.ds(off2, CHUNKB)], idx2_v.at[b], sx).wait()

    def do_chunk(ci, b, prefetch):
        off = ebase + ci * CHUNKB
        ins = [pltpu.async_copy(efn_h.at[pl.ds(off, CHUNKB)], ef_v, semi)]
        if prefetch:
            fire_idx(ci + 1, 1 - b)
        wait_idx(ci, b)
        cp = pltpu.async_copy(t23_h.at[idx2_v.at[b]], r1_v, sem)
        ins[0].wait()
        cp.wait()

        @plsc.parallel_loop(0, CHUNKB, unroll=4)
        def _rowc(r):
            for j in range(4):
                sl = pl.ds(16 * j, 16)
                sh = pl.ds(64 + 16 * j, 16)
                sg = 1.0 / (1.0 + jnp.exp(-ef_v[r, sl]))
                r1_v[r, sl] = sg * r1_v[r, sh]
                r1_v[r, sh] = sg
        for m in range(CHUNKB // 16):
            idx2d_v[0, pl.ds(16 * m, 16)] = idx1_v[b, pl.ds(16 * m, 16)]
        pltpu.sync_copy(r1_v, nbd_sh.at[idx2d_v.at[0]], add=True)

    fire_idx(0, 0)

    def pair_body(i, carry):
        do_chunk(2 * i, 0, True)
        do_chunk(2 * i + 1, 1, True)
        return carry

    lax.fori_loop(0, NCHUNKB // 2, pair_body, 0)
    do_chunk(NCHUNKB - 1, 0, False)
    plsc.subcore_barrier()

    @pl.when(sid == 0)
    def _():
        pltpu.sync_copy(nbd_sh, nbd_h.at[cid])


def _sc_gather2(src_h, dst_h, t_h, out_h,
                idx1_v, idx2_v, r1_v, r2_v, o_v, sem, semi, semx):
    wid = _wid()
    ebase = wid * EPW

    def chunk_body(ci, carry):
        off = ebase + ci * CHUNK
        ixs = [pltpu.async_copy(src_h.at[pl.ds(off, CHUNK)], idx1_v, semx),
               pltpu.async_copy(dst_h.at[pl.ds(off, CHUNK)], idx2_v, semx)]
        ixs[0].wait()
        ixs[1].wait()
        cps = []
        for k in range(KSUB):
            sl = pl.ds(SUB * k, SUB)
            cps.append(pltpu.async_copy(
                t_h.at[idx1_v.at[sl]], r1_v.at[sl], sem))
            cps.append(pltpu.async_copy(
                t_h.at[idx2_v.at[sl]], r2_v.at[sl], sem))
        for cp in cps:
            cp.wait()

        @plsc.parallel_loop(0, CHUNK, unroll=4)
        def _rowg(r):
            for j in range(4):
                sl = pl.ds(16 * j, 16)
                sh = pl.ds(64 + 16 * j, 16)
                o_v[r, sl] = jnp.maximum(r1_v[r, sl] + r2_v[r, sh], 0.0)
        pltpu.sync_copy(o_v, out_h.at[pl.ds(off, CHUNK)])
        return carry

    lax.fori_loop(0, NCHUNK, chunk_body, 0)


def _sc_gather2e(src_h, dst_h, t_h, ex_h, out_h,
                 idx1_v, idx2_v, r1_v, r2_v, ex_v, sem, semi, semx):
    wid = _wid()
    ebase = wid * EPW

    def chunk_body(ci, carry):
        off = ebase + ci * CHUNK
        ixs = [pltpu.async_copy(src_h.at[pl.ds(off, CHUNK)], idx1_v, semx),
               pltpu.async_copy(dst_h.at[pl.ds(off, CHUNK)], idx2_v, semx)]
        ins = [pltpu.async_copy(ex_h.at[pl.ds(off, CHUNK)], ex_v, semi)]
        ixs[0].wait()
        ixs[1].wait()
        cps = []
        for k in range(KSUB):
            sl = pl.ds(SUB * k, SUB)
            cps.append(pltpu.async_copy(
                t_h.at[idx1_v.at[sl]], r1_v.at[sl], sem))
            cps.append(pltpu.async_copy(
                t_h.at[idx2_v.at[sl]], r2_v.at[sl], sem))
        ins[0].wait()
        for cp in cps:
            cp.wait()

        @plsc.parallel_loop(0, CHUNK, unroll=4)
        def _rowge(r):
            for j in range(4):
                sl = pl.ds(16 * j, 16)
                sh = pl.ds(64 + 16 * j, 16)
                ex_v[r, sl] = jnp.maximum(
                    r1_v[r, sl] + r2_v[r, sh] + ex_v[r, sl], 0.0)
        pltpu.sync_copy(ex_v, out_h.at[pl.ds(off, CHUNK)])
        return carry

    lax.fori_loop(0, NCHUNK, chunk_body, 0)


def _sds(shape):
    return jax.ShapeDtypeStruct(shape, _f32)


_IDX = pltpu.VMEM((CHUNK,), jnp.int32)
_ROWS = pltpu.VMEM((CHUNK, D2), _f32)
_HALF = pltpu.VMEM((CHUNK, D), _f32)
_IDXB = pltpu.VMEM((CHUNKB,), jnp.int32)
_IDXB2 = pltpu.VMEM((2, CHUNKB), jnp.int32)
_IDX2DB = pltpu.VMEM((1, CHUNKB), jnp.int32)
_ROWSB = pltpu.VMEM((CHUNKB, D2), _f32)
_HALFB = pltpu.VMEM((CHUNKB, D), _f32)


def _sc_a(src, dst, b3e, t12):
    return pl.kernel(
        _sc_pass_a,
        out_type=[_sds((E, D)), _sds((NW, 2, D))],
        mesh=_mesh,
        scratch_types=[_IDX, _IDX, _ROWS, _ROWS, _HALF,
                       pltpu.VMEM((2, D), _f32),
                       pltpu.SemaphoreType.DMA,
                       pltpu.SemaphoreType.DMA,
                       pltpu.SemaphoreType.DMA],
    )(src, dst, b3e, t12)


def _sc_b(src, dst, etmp, ef, t23, st, zero):
    return pl.kernel(
        _sc_pass_b,
        out_type=[_sds((E, D)), _sds((NC, N, D2))],
        mesh=_mesh,
        scratch_types=[_IDXB2, _IDXB2, _IDX2DB, _ROWSB, _HALFB, _HALFB,
                       pltpu.VMEM((2, D), _f32),
                       pltpu.VMEM_SHARED((N, D2), _f32),
                       pltpu.SemaphoreType.DMA,
                       pltpu.SemaphoreType.DMA,
                       pltpu.SemaphoreType.DMA,
                       pltpu.SemaphoreType.DMA],
    )(src, dst, etmp, ef, t23, st, zero)


def _sc_c(src, dst, efn, t23, zero):
    return pl.kernel(
        _sc_pass_c,
        out_type=_sds((NC, N, D2)),
        mesh=_mesh,
        scratch_types=[_IDXB2, _IDXB2, _IDX2DB, _ROWSB, _HALFB,
                       pltpu.VMEM_SHARED((N, D2), _f32),
                       pltpu.SemaphoreType.DMA,
                       pltpu.SemaphoreType.DMA,
                       pltpu.SemaphoreType.DMA,
                       pltpu.SemaphoreType.DMA],
    )(src, dst, efn, t23, zero)


def _sc_g2(src, dst, t):
    return pl.kernel(
        _sc_gather2,
        out_type=_sds((E, D)),
        mesh=_mesh,
        scratch_types=[_IDX, _IDX, _ROWS, _ROWS, _HALF,
                       pltpu.SemaphoreType.DMA,
                       pltpu.SemaphoreType.DMA,
                       pltpu.SemaphoreType.DMA],
    )(src, dst, t)


def _sc_g2e(src, dst, t, ex):
    return pl.kernel(
        _sc_gather2e,
        out_type=_sds((E, D)),
        mesh=_mesh,
        scratch_types=[_IDX, _IDX, _ROWS, _ROWS, _HALF,
                       pltpu.SemaphoreType.DMA,
                       pltpu.SemaphoreType.DMA,
                       pltpu.SemaphoreType.DMA],
    )(src, dst, t, ex)


def _tc_node_mlp_body(x_ref, w1_ref, b1_ref, w2_ref, b2_ref, o_ref):
    hh = jnp.maximum(
        jnp.dot(x_ref[...], w1_ref[...],
                preferred_element_type=_f32) + b1_ref[...], 0.0)
    o_ref[...] = jnp.dot(hh, w2_ref[...],
                         preferred_element_type=_f32) + b2_ref[...]


def _tc_node_mlp(x, W1n, b1n, W2n, b2n):
    return pl.pallas_call(
        _tc_node_mlp_body,
        out_shape=_sds((N, D)),
    )(x, W1n, b1n[None], W2n, b2n[None])


def _tc_pair2_body(a_ref, b_ref, w_ref, bias_ref, o_ref):
    a, b = a_ref[...], b_ref[...]
    o_ref[:, 0:D] = (jnp.dot(a, w_ref[0], preferred_element_type=_f32)
                     + jnp.dot(b, w_ref[1], preferred_element_type=_f32)
                     + bias_ref[0, 0])
    o_ref[:, D:D2] = (jnp.dot(a, w_ref[2], preferred_element_type=_f32)
                      + jnp.dot(b, w_ref[3], preferred_element_type=_f32)
                      + bias_ref[0, 1])


def _tc_pair2(a, b, w4, bias2):
    return pl.pallas_call(
        _tc_pair2_body,
        out_shape=_sds((N, D2)),
    )(a, b, w4, bias2[None])


def _tc_mm_body(relu, x_ref, w_ref, b_ref, o_ref):
    y = jnp.dot(x_ref[...], w_ref[...],
                preferred_element_type=_f32) + b_ref[...]
    o_ref[...] = jnp.maximum(y, 0.0) if relu else y


def _tc_mm(x, w, b, relu, blk=8000):
    rows, din = x.shape
    dout = w.shape[1]
    return pl.pallas_call(
        functools.partial(_tc_mm_body, relu),
        grid=(rows // blk,),
        in_specs=[
            pl.BlockSpec((blk, din), lambda i: (i, 0)),
            pl.BlockSpec((din, dout), lambda i: (0, 0)),
            pl.BlockSpec((1, dout), lambda i: (0, 0)),
        ],
        out_specs=pl.BlockSpec((blk, dout), lambda i: (i, 0)),
        out_shape=_sds((rows, dout)),
    )(x, w, b[None])


def _tc_node5_body(h_ref, w_ref, b_ref, a1_ref, t12_ref, t23_ref):
    h = h_ref[...]
    mm = lambda i: jnp.dot(h, w_ref[i], preferred_element_type=_f32) \
        + b_ref[i, 0]
    a1_ref[...] = mm(0)
    t23_ref[:, 0:D] = mm(1)
    t23_ref[:, D:D2] = mm(2)
    t12_ref[:, 0:D] = mm(3)
    t12_ref[:, D:D2] = mm(4)


def _tc_node5(h, w5, b5):
    return pl.pallas_call(
        _tc_node5_body,
        out_shape=[_sds((N, D)), _sds((N, D2)), _sds((N, D2))],
    )(h, w5, b5)


def _tc_stats_body(s_ref, g_ref, b_ref, o_ref):
    tot = jnp.sum(s_ref[...], axis=0)
    mean = tot[0] / E
    var = tot[1] / E - mean * mean
    sv = g_ref[0] * lax.rsqrt(var + 1e-5)
    tv = b_ref[0] - mean * sv
    o_ref[...] = jnp.stack([sv, tv])


def _tc_stats(stats, g, b):
    return pl.pallas_call(
        _tc_stats_body,
        out_shape=_sds((2, D)),
    )(stats, g[None], b[None])


def _tc_hup_body(h_ref, a1_ref, nfd_ref, nbd_ref, g_ref, b_ref, o_ref):
    nf = nfd_ref[0, :, 0:D] + nfd_ref[1, :, 0:D]
    df = nfd_ref[0, :, D:D2] + nfd_ref[1, :, D:D2]
    nb = nbd_ref[0, :, 0:D] + nbd_ref[1, :, 0:D]
    db = nbd_ref[0, :, D:D2] + nbd_ref[1, :, D:D2]
    t = a1_ref[...] + nf / (df + 1e-6) + nb / (db + 1e-6)
    mu = jnp.mean(t, axis=0, keepdims=True)
    var = jnp.mean((t - mu) * (t - mu), axis=0, keepdims=True)
    bn = g_ref[...] * (t - mu) * lax.rsqrt(var + 1e-5) + b_ref[...]
    o_ref[...] = h_ref[...] + jnp.maximum(bn, 0.0)


def _tc_hup(h, a1, nfd, nbd, g, b):
    return pl.pallas_call(
        _tc_hup_body,
        out_shape=_sds((N, D)),
    )(h, a1, nfd, nbd, g[None], b[None])


_DBG_JAX_A = False
_DBG_JAX_B = False
_DBG_JAX_C = False
_DBG_JAX_G2 = False


def _jax_a(src, dst, b3e, t12):
    etmp = b3e + t12[:, 0:D][src] + t12[:, D:D2][dst]
    stats = jnp.stack([
        jnp.sum(etmp, axis=0), jnp.sum(etmp * etmp, axis=0)])[None]
    stats = jnp.concatenate([stats, jnp.zeros((NW - 1, 2, D), _f32)], 0)
    return etmp, stats


def _jax_b(src, dst, etmp, ef, t23, st, zero):
    efn = ef + jnp.maximum(etmp * st[0] + st[1], 0.0)
    sg = jax.nn.sigmoid(efn)
    nf = jax.ops.segment_sum(sg * t23[:, 0:D][src], dst, num_segments=N)
    df = jax.ops.segment_sum(sg, dst, num_segments=N)
    nfd = jnp.concatenate([nf, df], axis=1)[None]
    nfd = jnp.concatenate([nfd, jnp.zeros((1, N, D2), _f32)], 0)
    return efn, nfd


def _jax_c(src, dst, efn, t23, zero):
    sg = jax.nn.sigmoid(efn)
    nb = jax.ops.segment_sum(sg * t23[:, D:D2][dst], src, num_segments=N)
    db = jax.ops.segment_sum(sg, src, num_segments=N)
    nbd = jnp.concatenate([nb, db], axis=1)[None]
    return jnp.concatenate([nbd, jnp.zeros((1, N, D2), _f32)], 0)


def kernel(x, edge_index, e, W1n, b1n, W2n, b2n, W1e, b1e, W2e, b2e,
           gA1, bgA1, gA2, bgA2, gA3, bgA3, gB1, bgB1, gB2, bgB2, gB3, bgB3,
           gam_h, bet_h, gam_e, bet_e, pW1, pb1, pW2, pb2):
    src = edge_index[0]
    dst = edge_index[1]
    zero2 = jnp.zeros((N, D2), _f32)

    h = _tc_node_mlp(x, W1n, b1n, W2n, b2n)
    x2 = jax.random.normal(jax.random.key(1), (N, D), dtype=_f32)

    tuv = _tc_pair2(h, x2,
                    jnp.stack([W1e[0:64], W1e[128:192],
                               W1e[64:128], W1e[192:256]]),
                    jnp.stack([b1e, jnp.zeros((D,), _f32)]))
    he = _sc_g2(src, dst, tuv)
    ef = _tc_mm(he, W2e, b2e, relu=True)

    w5 = jnp.stack([gA1, gA2, gA3, gB1, gB2])
    b5 = jnp.stack([bgA1, bgA2, bgA3, bgB1, bgB2])[:, :, None, :]

    for l in range(L):
        a1t, t12, t23 = _tc_node5(h, w5[:, l], b5[:, l])
        b3e = _tc_mm(ef, gB3[l], bgB3[l], relu=False)
        etmp, stats = (_jax_a if _DBG_JAX_A else _sc_a)(src, dst, b3e, t12)
        st = _tc_stats(stats, gam_e[l], bet_e[l])
        efn, nfd = (_jax_b if _DBG_JAX_B else _sc_b)(
            src, dst, etmp, ef, t23, st, zero2)
        nbd = (_jax_c if _DBG_JAX_C else _sc_c)(src, dst, efn, t23, zero2)
        ef = efn
        h = _tc_hup(h, a1t, nfd, nbd, gam_h[l], bet_h[l])

    thp = _tc_pair2(h, h,
                    jnp.stack([pW1[0:64], jnp.zeros((D, D), _f32),
                               pW1[64:128], jnp.zeros((D, D), _f32)]),
                    jnp.stack([pb1, jnp.zeros((D,), _f32)]))
    efp3 = _tc_mm(ef, pW1[128:192], jnp.zeros((D,), _f32), relu=False)
    ph = _sc_g2e(src, dst, thp, efp3)
    scores = _tc_mm(ph, pW2, pb2, relu=False)
    return scores


# speedup vs baseline: 4.6647x; 1.0015x over previous
import functools

import jax
import jax.numpy as jnp
from jax import lax
from jax.experimental import pallas as pl
from jax.experimental.pallas import tpu as pltpu
from jax.experimental.pallas import tpu_sc as plsc

L = 8
N = 10000
E = 320000
D = 64
D2 = 128
NC = 2
NS = 16
NW = NC * NS
EPW = E // NW
SUB = 40
KSUB = 5
CHUNK = SUB * KSUB
NCHUNK = EPW // CHUNK
CHUNKB = 80
NCHUNKB = EPW // CHUNKB

_mesh = plsc.VectorSubcoreMesh(core_axis_name="c", subcore_axis_name="s")
_f32 = jnp.float32


def _wid():
    return lax.axis_index("s") * NC + lax.axis_index("c")


def _sc_pass_a(src_h, dst_h, b3e_h, t12_h, etmp_h, stats_h,
               idx1_v, idx2_v, r1_v, r2_v, acc_v, st_v, sem, semi, semx):
    wid = _wid()
    ebase = wid * EPW

    def chunk_body(ci, carry):
        off = ebase + ci * CHUNK
        ixs = [pltpu.async_copy(src_h.at[pl.ds(off, CHUNK)], idx1_v, semx),
               pltpu.async_copy(dst_h.at[pl.ds(off, CHUNK)], idx2_v, semx)]
        ins = [pltpu.async_copy(b3e_h.at[pl.ds(off, CHUNK)], acc_v, semi)]
        ixs[0].wait()
        ixs[1].wait()
        cps = []
        for k in range(KSUB):
            sl = pl.ds(SUB * k, SUB)
            cps.append(pltpu.async_copy(
                t12_h.at[idx1_v.at[sl]], r1_v.at[sl], sem))
            cps.append(pltpu.async_copy(
                t12_h.at[idx2_v.at[sl]], r2_v.at[sl], sem))
        ins[0].wait()
        for cp in cps:
            cp.wait()

        @plsc.parallel_loop(0, CHUNK, unroll=4, carry=tuple(carry))
        def row_sums(r, c2):
            sums = list(c2)
            for j in range(4):
                sl = pl.ds(16 * j, 16)
                sh = pl.ds(64 + 16 * j, 16)
                a = acc_v[r, sl] + r1_v[r, sl] + r2_v[r, sh]
                acc_v[r, sl] = a
                sums[j] = sums[j] + a
                sums[4 + j] = sums[4 + j] + a * a
            return tuple(sums)

        carry = row_sums
        pltpu.sync_copy(acc_v, etmp_h.at[pl.ds(off, CHUNK)])
        return carry

    z = jnp.zeros((16,), _f32)
    sums = lax.fori_loop(0, NCHUNK, chunk_body, (z,) * 8)
    for j in range(4):
        st_v[0, pl.ds(16 * j, 16)] = sums[j]
        st_v[1, pl.ds(16 * j, 16)] = sums[4 + j]
    pltpu.sync_copy(st_v, stats_h.at[wid])


def _sc_pass_b(src_h, dst_h, etmp_h, ef_h, t23_h, st_h, zero_h,
               efn_h, nfd_h,
               idx1_v, idx2_v, idx2d_v, r1_v, et_v, ef_v, st_v, nfd_sh, sem, semi, semx0, semx1):
    cid = lax.axis_index("c")
    sid = lax.axis_index("s")
    wid = sid * NC + cid
    ebase = wid * EPW

    @pl.when(sid == 0)
    def _():
        pltpu.sync_copy(zero_h, nfd_sh)
    plsc.subcore_barrier()

    pltpu.sync_copy(st_h, st_v)
    sv = [st_v[0, pl.ds(16 * j, 16)] for j in range(4)]
    tv = [st_v[1, pl.ds(16 * j, 16)] for j in range(4)]

    def fire_idx(ci, b):
        off2 = ebase + ci * CHUNKB
        sx = semx0 if b == 0 else semx1
        pltpu.async_copy(src_h.at[pl.ds(off2, CHUNKB)], idx1_v.at[b], sx)
        pltpu.async_copy(dst_h.at[pl.ds(off2, CHUNKB)], idx2_v.at[b], sx)

    def wait_idx(ci, b):
        off2 = ebase + ci * CHUNKB
        sx = semx0 if b == 0 else semx1
        pltpu.make_async_copy(
            src_h.at[pl.ds(off2, CHUNKB)], idx1_v.at[b], sx).wait()
        pltpu.make_async_copy(
            dst_h.at[pl.ds(off2, CHUNKB)], idx2_v.at[b], sx).wait()

    def do_chunk(ci, b, prefetch):
        off = ebase + ci * CHUNKB
        ins = [pltpu.async_copy(etmp_h.at[pl.ds(off, CHUNKB)], et_v, semi),
               pltpu.async_copy(ef_h.at[pl.ds(off, CHUNKB)], ef_v, semi)]
        if prefetch:
            fire_idx(ci + 1, 1 - b)
        wait_idx(ci, b)
        cp = pltpu.async_copy(t23_h.at[idx1_v.at[b]], r1_v, sem)
        ins[0].wait()
        ins[1].wait()
        cp.wait()

        @plsc.parallel_loop(0, CHUNKB, unroll=4)
        def _rowb(r):
            for j in range(4):
                sl = pl.ds(16 * j, 16)
                sh = pl.ds(64 + 16 * j, 16)
                x = jnp.maximum(et_v[r, sl] * sv[j] + tv[j], 0.0) + ef_v[r, sl]
                ef_v[r, sl] = x
                sg = 1.0 / (1.0 + jnp.exp(-x))
                r1_v[r, sl] = sg * r1_v[r, sl]
                r1_v[r, sh] = sg
        pltpu.sync_copy(ef_v, efn_h.at[pl.ds(off, CHUNKB)])
        for m in range(CHUNKB // 16):
            idx2d_v[0, pl.ds(16 * m, 16)] = idx2_v[b, pl.ds(16 * m, 16)]
        pltpu.sync_copy(r1_v, nfd_sh.at[idx2d_v.at[0]], add=True)

    fire_idx(0, 0)

    def pair_body(i, carry):
        do_chunk(2 * i, 0, True)
        do_chunk(2 * i + 1, 1, True)
        return carry

    lax.fori_loop(0, NCHUNKB // 2, pair_body, 0)
    do_chunk(NCHUNKB - 1, 0, False)
    plsc.subcore_barrier()

    @pl.when(sid == 0)
    def _():
        pltpu.sync_copy(nfd_sh, nfd_h.at[cid])


def _sc_pass_c(src_h, dst_h, efn_h, t23_h, zero_h,
               nbd_h,
               idx1_v, idx2_v, idx2d_v, r1_v, ef_v, nbd_sh, sem, semi, semx0, semx1):
    cid = lax.axis_index("c")
    sid = lax.axis_index("s")
    wid = sid * NC + cid
    ebase = wid * EPW

    @pl.when(sid == 0)
    def _():
        pltpu.sync_copy(zero_h, nbd_sh)
    plsc.subcore_barrier()

    def fire_idx(ci, b):
        off2 = ebase + ci * CHUNKB
        sx = semx0 if b == 0 else semx1
        pltpu.async_copy(src_h.at[pl.ds(off2, CHUNKB)], idx1_v.at[b], sx)
        pltpu.async_copy(dst_h.at[pl.ds(off2, CHUNKB)], idx2_v.at[b], sx)

    def wait_idx(ci, b):
        off2 = ebase + ci * CHUNKB
        sx = semx0 if b == 0 else semx1
        pltpu.make_async_copy(
            src_h.at[pl.ds(off2, CHUNKB)], idx1_v.at[b], sx).wait()
        pltpu.make_async_copy(
            dst_h.at[pl.ds(off2, CHUNKB)], idx2_v.at[b], sx).wait()

    def do_chunk(ci, b, prefetch):
        off = ebase + ci * CHUNKB
        ins = [pltpu.async_copy(efn_h.at[pl.ds(off, CHUNKB)], ef_v, semi)]
        if prefetch:
            fire_idx(ci + 1, 1 - b)
        wait_idx(ci, b)
        cp = pltpu.async_copy(t23_h.at[idx2_v.at[b]], r1_v, sem)
        ins[0].wait()
        cp.wait()

        @plsc.parallel_loop(0, CHUNKB, unroll=4)
        def _rowc(r):
            for j in range(4):
                sl = pl.ds(16 * j, 16)
                sh = pl.ds(64 + 16 * j, 16)
                sg = 1.0 / (1.0 + jnp.exp(-ef_v[r, sl]))
                r1_v[r, sl] = sg * r1_v[r, sh]
                r1_v[r, sh] = sg
        for m in range(CHUNKB // 16):
            idx2d_v[0, pl.ds(16 * m, 16)] = idx1_v[b, pl.ds(16 * m, 16)]
        pltpu.sync_copy(r1_v, nbd_sh.at[idx2d_v.at[0]], add=True)

    fire_idx(0, 0)

    def pair_body(i, carry):
        do_chunk(2 * i, 0, True)
        do_chunk(2 * i + 1, 1, True)
        return carry

    lax.fori_loop(0, NCHUNKB // 2, pair_body, 0)
    do_chunk(NCHUNKB - 1, 0, False)
    plsc.subcore_barrier()

    @pl.when(sid == 0)
    def _():
        pltpu.sync_copy(nbd_sh, nbd_h.at[cid])


def _sc_gather2(src_h, dst_h, t_h, out_h,
                idx1_v, idx2_v, r1_v, r2_v, o_v, sem, semi, semx):
    wid = _wid()
    ebase = wid * EPW

    def chunk_body(ci, carry):
        off = ebase + ci * CHUNK
        ixs = [pltpu.async_copy(src_h.at[pl.ds(off, CHUNK)], idx1_v, semx),
               pltpu.async_copy(dst_h.at[pl.ds(off, CHUNK)], idx2_v, semx)]
        ixs[0].wait()
        ixs[1].wait()
        cps = []
        for k in range(KSUB):
            sl = pl.ds(SUB * k, SUB)
            cps.append(pltpu.async_copy(
                t_h.at[idx1_v.at[sl]], r1_v.at[sl], sem))
            cps.append(pltpu.async_copy(
                t_h.at[idx2_v.at[sl]], r2_v.at[sl], sem))
        for cp in cps:
            cp.wait()

        @plsc.parallel_loop(0, CHUNK, unroll=4)
        def _rowg(r):
            for j in range(4):
                sl = pl.ds(16 * j, 16)
                sh = pl.ds(64 + 16 * j, 16)
                o_v[r, sl] = jnp.maximum(r1_v[r, sl] + r2_v[r, sh], 0.0)
        pltpu.sync_copy(o_v, out_h.at[pl.ds(off, CHUNK)])
        return carry

    lax.fori_loop(0, NCHUNK, chunk_body, 0)


def _sc_gather2e(src_h, dst_h, t_h, ex_h, out_h,
                 idx1_v, idx2_v, r1_v, r2_v, ex_v, sem, semi, semx):
    wid = _wid()
    ebase = wid * EPW

    def chunk_body(ci, carry):
        off = ebase + ci * CHUNK
        ixs = [pltpu.async_copy(src_h.at[pl.ds(off, CHUNK)], idx1_v, semx),
               pltpu.async_copy(dst_h.at[pl.ds(off, CHUNK)], idx2_v, semx)]
        ins = [pltpu.async_copy(ex_h.at[pl.ds(off, CHUNK)], ex_v, semi)]
        ixs[0].wait()
        ixs[1].wait()
        cps = []
        for k in range(KSUB):
            sl = pl.ds(SUB * k, SUB)
            cps.append(pltpu.async_copy(
                t_h.at[idx1_v.at[sl]], r1_v.at[sl], sem))
            cps.append(pltpu.async_copy(
                t_h.at[idx2_v.at[sl]], r2_v.at[sl], sem))
        ins[0].wait()
        for cp in cps:
            cp.wait()

        @plsc.parallel_loop(0, CHUNK, unroll=4)
        def _rowge(r):
            for j in range(4):
                sl = pl.ds(16 * j, 16)
                sh = pl.ds(64 + 16 * j, 16)
                ex_v[r, sl] = jnp.maximum(
                    r1_v[r, sl] + r2_v[r, sh] + ex_v[r, sl], 0.0)
        pltpu.sync_copy(ex_v, out_h.at[pl.ds(off, CHUNK)])
        return carry

    lax.fori_loop(0, NCHUNK, chunk_body, 0)


def _sds(shape):
    return jax.ShapeDtypeStruct(shape, _f32)


_IDX = pltpu.VMEM((CHUNK,), jnp.int32)
_ROWS = pltpu.VMEM((CHUNK, D2), _f32)
_HALF = pltpu.VMEM((CHUNK, D), _f32)
_IDXB = pltpu.VMEM((CHUNKB,), jnp.int32)
_IDXB2 = pltpu.VMEM((2, CHUNKB), jnp.int32)
_IDX2DB = pltpu.VMEM((1, CHUNKB), jnp.int32)
_ROWSB = pltpu.VMEM((CHUNKB, D2), _f32)
_HALFB = pltpu.VMEM((CHUNKB, D), _f32)


def _sc_a(src, dst, b3e, t12):
    return pl.kernel(
        _sc_pass_a,
        out_type=[_sds((E, D)), _sds((NW, 2, D))],
        mesh=_mesh,
        scratch_types=[_IDX, _IDX, _ROWS, _ROWS, _HALF,
                       pltpu.VMEM((2, D), _f32),
                       pltpu.SemaphoreType.DMA,
                       pltpu.SemaphoreType.DMA,
                       pltpu.SemaphoreType.DMA],
    )(src, dst, b3e, t12)


def _sc_b(src, dst, etmp, ef, t23, st, zero):
    return pl.kernel(
        _sc_pass_b,
        out_type=[_sds((E, D)), _sds((NC, N, D2))],
        mesh=_mesh,
        scratch_types=[_IDXB2, _IDXB2, _IDX2DB, _ROWSB, _HALFB, _HALFB,
                       pltpu.VMEM((2, D), _f32),
                       pltpu.VMEM_SHARED((N, D2), _f32),
                       pltpu.SemaphoreType.DMA,
                       pltpu.SemaphoreType.DMA,
                       pltpu.SemaphoreType.DMA,
                       pltpu.SemaphoreType.DMA],
    )(src, dst, etmp, ef, t23, st, zero)


def _sc_c(src, dst, efn, t23, zero):
    return pl.kernel(
        _sc_pass_c,
        out_type=_sds((NC, N, D2)),
        mesh=_mesh,
        scratch_types=[_IDXB2, _IDXB2, _IDX2DB, _ROWSB, _HALFB,
                       pltpu.VMEM_SHARED((N, D2), _f32),
                       pltpu.SemaphoreType.DMA,
                       pltpu.SemaphoreType.DMA,
                       pltpu.SemaphoreType.DMA,
                       pltpu.SemaphoreType.DMA],
    )(src, dst, efn, t23, zero)


def _sc_g2(src, dst, t):
    return pl.kernel(
        _sc_gather2,
        out_type=_sds((E, D)),
        mesh=_mesh,
        scratch_types=[_IDX, _IDX, _ROWS, _ROWS, _HALF,
                       pltpu.SemaphoreType.DMA,
                       pltpu.SemaphoreType.DMA,
                       pltpu.SemaphoreType.DMA],
    )(src, dst, t)


def _sc_g2e(src, dst, t, ex):
    return pl.kernel(
        _sc_gather2e,
        out_type=_sds((E, D)),
        mesh=_mesh,
        scratch_types=[_IDX, _IDX, _ROWS, _ROWS, _HALF,
                       pltpu.SemaphoreType.DMA,
                       pltpu.SemaphoreType.DMA,
                       pltpu.SemaphoreType.DMA],
    )(src, dst, t, ex)


def _tc_node_mlp_body(x_ref, w1_ref, b1_ref, w2_ref, b2_ref, o_ref):
    hh = jnp.maximum(
        jnp.dot(x_ref[...], w1_ref[...],
                preferred_element_type=_f32) + b1_ref[...], 0.0)
    o_ref[...] = jnp.dot(hh, w2_ref[...],
                         preferred_element_type=_f32) + b2_ref[...]


def _tc_node_mlp(x, W1n, b1n, W2n, b2n):
    return pl.pallas_call(
        _tc_node_mlp_body,
        out_shape=_sds((N, D)),
    )(x, W1n, b1n[None], W2n, b2n[None])


def _tc_pair2_body(a_ref, b_ref, w_ref, bias_ref, o_ref):
    a, b = a_ref[...], b_ref[...]
    o_ref[:, 0:D] = (jnp.dot(a, w_ref[0], preferred_element_type=_f32)
                     + jnp.dot(b, w_ref[1], preferred_element_type=_f32)
                     + bias_ref[0, 0])
    o_ref[:, D:D2] = (jnp.dot(a, w_ref[2], preferred_element_type=_f32)
                      + jnp.dot(b, w_ref[3], preferred_element_type=_f32)
                      + bias_ref[0, 1])


def _tc_pair2(a, b, w4, bias2):
    return pl.pallas_call(
        _tc_pair2_body,
        out_shape=_sds((N, D2)),
    )(a, b, w4, bias2[None])


def _tc_mm_body(relu, x_ref, w_ref, b_ref, o_ref):
    y = jnp.dot(x_ref[...], w_ref[...],
                preferred_element_type=_f32) + b_ref[...]
    o_ref[...] = jnp.maximum(y, 0.0) if relu else y


def _tc_mm(x, w, b, relu, blk=8000):
    rows, din = x.shape
    dout = w.shape[1]
    return pl.pallas_call(
        functools.partial(_tc_mm_body, relu),
        grid=(rows // blk,),
        in_specs=[
            pl.BlockSpec((blk, din), lambda i: (i, 0)),
            pl.BlockSpec((din, dout), lambda i: (0, 0)),
            pl.BlockSpec((1, dout), lambda i: (0, 0)),
        ],
        out_specs=pl.BlockSpec((blk, dout), lambda i: (i, 0)),
        out_shape=_sds((rows, dout)),
    )(x, w, b[None])


def _tc_node5_body(h_ref, w_ref, b_ref, a1_ref, t12_ref, t23_ref):
    h = h_ref[...]
    mm = lambda i: jnp.dot(h, w_ref[i], preferred_element_type=_f32) \
        + b_ref[i, 0]
    a1_ref[...] = mm(0)
    t23_ref[:, 0:D] = mm(1)
    t23_ref[:, D:D2] = mm(2)
    t12_ref[:, 0:D] = mm(3)
    t12_ref[:, D:D2] = mm(4)


def _tc_node5(h, w5, b5):
    return pl.pallas_call(
        _tc_node5_body,
        out_shape=[_sds((N, D)), _sds((N, D2)), _sds((N, D2))],
    )(h, w5, b5)


def _tc_stats_body(s_ref, g_ref, b_ref, o_ref):
    tot = jnp.sum(s_ref[...], axis=0)
    mean = tot[0] / E
    var = tot[1] / E - mean * mean
    sv = g_ref[0] * lax.rsqrt(var + 1e-5)
    tv = b_ref[0] - mean * sv
    o_ref[...] = jnp.stack([sv, tv])


def _tc_stats(stats, g, b):
    return pl.pallas_call(
        _tc_stats_body,
        out_shape=_sds((2, D)),
    )(stats, g[None], b[None])


def _tc_hup_body(h_ref, a1_ref, nfd_ref, nbd_ref, g_ref, b_ref, o_ref):
    nf = nfd_ref[0, :, 0:D] + nfd_ref[1, :, 0:D]
    df = nfd_ref[0, :, D:D2] + nfd_ref[1, :, D:D2]
    nb = nbd_ref[0, :, 0:D] + nbd_ref[1, :, 0:D]
    db = nbd_ref[0, :, D:D2] + nbd_ref[1, :, D:D2]
    t = a1_ref[...] + nf / (df + 1e-6) + nb / (db + 1e-6)
    mu = jnp.mean(t, axis=0, keepdims=True)
    var = jnp.mean((t - mu) * (t - mu), axis=0, keepdims=True)
    bn = g_ref[...] * (t - mu) * lax.rsqrt(var + 1e-5) + b_ref[...]
    o_ref[...] = h_ref[...] + jnp.maximum(bn, 0.0)


def _tc_hup(h, a1, nfd, nbd, g, b):
    return pl.pallas_call(
        _tc_hup_body,
        out_shape=_sds((N, D)),
    )(h, a1, nfd, nbd, g[None], b[None])


def kernel(x, edge_index, e, W1n, b1n, W2n, b2n, W1e, b1e, W2e, b2e,
           gA1, bgA1, gA2, bgA2, gA3, bgA3, gB1, bgB1, gB2, bgB2, gB3, bgB3,
           gam_h, bet_h, gam_e, bet_e, pW1, pb1, pW2, pb2):
    src = edge_index[0]
    dst = edge_index[1]
    zero2 = jnp.zeros((N, D2), _f32)

    h = _tc_node_mlp(x, W1n, b1n, W2n, b2n)
    x2 = jax.random.normal(jax.random.key(1), (N, D), dtype=_f32)

    tuv = _tc_pair2(h, x2,
                    jnp.stack([W1e[0:64], W1e[128:192],
                               W1e[64:128], W1e[192:256]]),
                    jnp.stack([b1e, jnp.zeros((D,), _f32)]))
    he = _sc_g2(src, dst, tuv)
    ef = _tc_mm(he, W2e, b2e, relu=True)

    w5 = jnp.stack([gA1, gA2, gA3, gB1, gB2])
    b5 = jnp.stack([bgA1, bgA2, bgA3, bgB1, bgB2])[:, :, None, :]

    for l in range(L):
        a1t, t12, t23 = _tc_node5(h, w5[:, l], b5[:, l])
        b3e = _tc_mm(ef, gB3[l], bgB3[l], relu=False)
        etmp, stats = _sc_a(src, dst, b3e, t12)
        st = _tc_stats(stats, gam_e[l], bet_e[l])
        efn, nfd = _sc_b(src, dst, etmp, ef, t23, st, zero2)
        nbd = _sc_c(src, dst, efn, t23, zero2)
        ef = efn
        h = _tc_hup(h, a1t, nfd, nbd, gam_h[l], bet_h[l])

    thp = _tc_pair2(h, h,
                    jnp.stack([pW1[0:64], jnp.zeros((D, D), _f32),
                               pW1[64:128], jnp.zeros((D, D), _f32)]),
                    jnp.stack([pb1, jnp.zeros((D,), _f32)]))
    efp3 = _tc_mm(ef, pW1[128:192], jnp.zeros((D,), _f32), relu=False)
    ph = _sc_g2e(src, dst, thp, efp3)
    scores = _tc_mm(ph, pW2, pb2, relu=False)
    return scores
